# Initial kernel scaffold; baseline (speedup 1.0000x reference)
import jax
import jax.numpy as jnp
from jax.experimental import pallas as pl


def kernel(h, he, edge_index, eW1, eb1, eW2, eb2, eln_g, eln_b, nWm1, nbm1, nWm2, nbm2, nln_g, nln_b, nWu1, nbu1, nWu2, nbu2, pn_g, pn_b, lin_W, lin_b, mln_g, mln_b):
    raise NotImplementedError("write your pallas kernel here")


# speedup vs baseline: 3.2356x; 3.2356x over previous
import functools

import jax
import jax.numpy as jnp
from jax import lax
from jax.experimental import pallas as pl
from jax.experimental.pallas import tpu as pltpu
from jax.experimental.pallas import tpu_sc as plsc

_N = 10000
_E = 160000
_D = 128
_DE = 16

_NC = 2
_NS = 16
_NW = _NC * _NS

_CH = 128
_NCHUNK = _E // _CH
_BASE_CH = _NCHUNK // _NW
_EXTRA = _NCHUNK - _BASE_CH * _NW


@functools.cache
def _sc_mesh():
    return plsc.VectorSubcoreMesh(
        core_axis_name="c", subcore_axis_name="s",
        num_cores=_NC, num_subcores=_NS)


_WB = 80
_NBLK = _N // _WB
_BLK_ROUNDS = (_NBLK + _NS - 1) // _NS


def _for_sub_blocks(s, fn):
    for kk in range(_BLK_ROUNDS):
        blk = s + _NS * kk

        @pl.when(blk < _NBLK)
        def _():
            fn(blk * _WB)


def _worker_id():
    c = lax.axis_index("c")
    s = lax.axis_index("s")
    return s * _NC + c, c, s


def _num_chunks(wid):
    return jnp.where(wid < _EXTRA, _BASE_CH + 1, _BASE_CH)


def _zero_vmem(ref, rows, cols):
    z = jnp.zeros((16,), jnp.float32)

    def body(r, cy):
        for j in range(cols // 16):
            ref[r, pl.ds(j * 16, 16)] = z
        return cy

    lax.fori_loop(0, rows, body, 0)


def _sc_gather_deg_body(h_hbm, src_hbm, dst_hbm, hs_hbm, hd_hbm, deg_hbm,
                        acc_sh, idx_v, rows_v, ones_v, zb_v, sem):
    wid, c, s = _worker_id()
    _zero_vmem(zb_v, _WB, _D)
    _for_sub_blocks(s, lambda off: pltpu.sync_copy(zb_v, acc_sh.at[pl.ds(off, _WB)]))

    def fill_ones(r, cy):
        one = jnp.ones((16,), jnp.float32)
        for j in range(_D // 16):
            ones_v[r, pl.ds(j * 16, 16)] = one
        return cy

    lax.fori_loop(0, _CH, fill_ones, 0)
    plsc.subcore_barrier()

    def body(i, cy):
        base = (wid + _NW * i) * _CH
        pltpu.sync_copy(src_hbm.at[pl.ds(base, _CH)], idx_v)
        pltpu.async_copy(h_hbm.at[idx_v], rows_v, sem).wait()
        pltpu.sync_copy(rows_v, hs_hbm.at[pl.ds(base, _CH)])
        pltpu.sync_copy(dst_hbm.at[pl.ds(base, _CH)], idx_v)
        pltpu.async_copy(h_hbm.at[idx_v], rows_v, sem).wait()
        pltpu.sync_copy(rows_v, hd_hbm.at[pl.ds(base, _CH)])
        pltpu.sync_copy(ones_v, acc_sh.at[idx_v], add=True)
        return cy

    lax.fori_loop(0, _num_chunks(wid), body, 0)
    plsc.subcore_barrier()

    def wb(off):
        pltpu.sync_copy(acc_sh.at[pl.ds(off, _WB)], zb_v)
        pltpu.sync_copy(zb_v, deg_hbm.at[c, pl.ds(off, _WB)])

    _for_sub_blocks(s, wb)


def _sc_gather_body(h_hbm, src_hbm, dst_hbm, hs_hbm, hd_hbm, idx_v, rows_v, sem):
    wid, _, _ = _worker_id()

    def body(i, cy):
        base = (wid + _NW * i) * _CH
        pltpu.sync_copy(src_hbm.at[pl.ds(base, _CH)], idx_v)
        pltpu.async_copy(h_hbm.at[idx_v], rows_v, sem).wait()
        pltpu.sync_copy(rows_v, hs_hbm.at[pl.ds(base, _CH)])
        pltpu.sync_copy(dst_hbm.at[pl.ds(base, _CH)], idx_v)
        pltpu.async_copy(h_hbm.at[idx_v], rows_v, sem).wait()
        pltpu.sync_copy(rows_v, hd_hbm.at[pl.ds(base, _CH)])
        return cy

    lax.fori_loop(0, _num_chunks(wid), body, 0)


@functools.cache
def _sc_gather(with_deg):
    if with_deg:
        return pl.kernel(
            _sc_gather_deg_body,
            out_type=(jax.ShapeDtypeStruct((_E, _D), jnp.float32),
                      jax.ShapeDtypeStruct((_E, _D), jnp.float32),
                      jax.ShapeDtypeStruct((_NC, _N, _D), jnp.float32)),
            mesh=_sc_mesh(),
            scratch_types=[
                pltpu.VMEM_SHARED((_N, _D), jnp.float32),
                pltpu.VMEM((_CH,), jnp.int32),
                pltpu.VMEM((_CH, _D), jnp.float32),
                pltpu.VMEM((_CH, _D), jnp.float32),
                pltpu.VMEM((_WB, _D), jnp.float32),
                pltpu.SemaphoreType.DMA,
            ],
        )
    return pl.kernel(
        _sc_gather_body,
        out_type=(jax.ShapeDtypeStruct((_E, _D), jnp.float32),
                  jax.ShapeDtypeStruct((_E, _D), jnp.float32)),
        mesh=_sc_mesh(),
        scratch_types=[
            pltpu.VMEM((_CH,), jnp.int32),
            pltpu.VMEM((_CH, _D), jnp.float32),
            pltpu.SemaphoreType.DMA,
        ],
    )


def _sc_segsum_body(vals_hbm, dst_hbm, out_hbm, acc_sh, rows_v, idx_v, zb_v):
    wid, c, s = _worker_id()
    _zero_vmem(zb_v, _WB, _D)
    _for_sub_blocks(s, lambda off: pltpu.sync_copy(zb_v, acc_sh.at[pl.ds(off, _WB)]))
    plsc.subcore_barrier()

    def body(i, cy):
        base = (wid + _NW * i) * _CH
        pltpu.sync_copy(vals_hbm.at[pl.ds(base, _CH)], rows_v)
        pltpu.sync_copy(dst_hbm.at[pl.ds(base, _CH)], idx_v)
        pltpu.sync_copy(rows_v, acc_sh.at[idx_v], add=True)
        return cy

    lax.fori_loop(0, _num_chunks(wid), body, 0)
    plsc.subcore_barrier()

    def wb(off):
        pltpu.sync_copy(acc_sh.at[pl.ds(off, _WB)], zb_v)
        pltpu.sync_copy(zb_v, out_hbm.at[c, pl.ds(off, _WB)])

    _for_sub_blocks(s, wb)


@functools.cache
def _sc_segsum():
    return pl.kernel(
        _sc_segsum_body,
        out_type=jax.ShapeDtypeStruct((_NC, _N, _D), jnp.float32),
        mesh=_sc_mesh(),
        scratch_types=[
            pltpu.VMEM_SHARED((_N, _D), jnp.float32),
            pltpu.VMEM((_CH, _D), jnp.float32),
            pltpu.VMEM((_CH,), jnp.int32),
            pltpu.VMEM((_WB, _D), jnp.float32),
        ],
    )


def _sc_diff_body(u_hbm, src_hbm, dst_hbm, out_hbm, acc_sh,
                  us_v, ud_v, si_v, di_v, zb_v, sem):
    wid, c, s = _worker_id()
    _zero_vmem(zb_v, _WB, _D)
    _for_sub_blocks(s, lambda off: pltpu.sync_copy(zb_v, acc_sh.at[pl.ds(off, _WB)]))
    plsc.subcore_barrier()

    def body(i, cy):
        base = (wid + _NW * i) * _CH
        pltpu.sync_copy(src_hbm.at[pl.ds(base, _CH)], si_v)
        pltpu.sync_copy(dst_hbm.at[pl.ds(base, _CH)], di_v)
        pltpu.async_copy(u_hbm.at[si_v], us_v, sem).wait()
        pltpu.async_copy(u_hbm.at[di_v], ud_v, sem).wait()

        def row(r, rcy):
            for j in range(_D // 16):
                a = us_v[r, pl.ds(j * 16, 16)]
                b = ud_v[r, pl.ds(j * 16, 16)]
                d = a - b
                us_v[r, pl.ds(j * 16, 16)] = d * d
            return rcy

        lax.fori_loop(0, _CH, row, 0)
        pltpu.sync_copy(us_v, acc_sh.at[di_v], add=True)
        return cy

    lax.fori_loop(0, _num_chunks(wid), body, 0)
    plsc.subcore_barrier()

    def wb(off):
        pltpu.sync_copy(acc_sh.at[pl.ds(off, _WB)], zb_v)
        pltpu.sync_copy(zb_v, out_hbm.at[c, pl.ds(off, _WB)])

    _for_sub_blocks(s, wb)


@functools.cache
def _sc_diff():
    return pl.kernel(
        _sc_diff_body,
        out_type=jax.ShapeDtypeStruct((_NC, _N, _D), jnp.float32),
        mesh=_sc_mesh(),
        scratch_types=[
            pltpu.VMEM_SHARED((_N, _D), jnp.float32),
            pltpu.VMEM((_CH, _D), jnp.float32),
            pltpu.VMEM((_CH, _D), jnp.float32),
            pltpu.VMEM((_CH,), jnp.int32),
            pltpu.VMEM((_CH,), jnp.int32),
            pltpu.VMEM((_WB, _D), jnp.float32),
            pltpu.SemaphoreType.DMA,
        ],
    )


_BE = 2000


def _em_kernel(hs, hd, he, eW1, eb1, eW2, eb2, eg, eb_, nW1, nb1, nW2, nb2,
               ng, nb_, he_out, m_out):
    a = hs[...]
    b = hd[...]
    cc = he[...]
    inv = 1.0 / (2 * _D + _DE)
    mu = (jnp.sum(a, 1, keepdims=True) + jnp.sum(b, 1, keepdims=True)
          + jnp.sum(cc, 1, keepdims=True)) * inv
    am = a - mu
    bm = b - mu
    cm = cc - mu
    var = (jnp.sum(am * am, 1, keepdims=True) + jnp.sum(bm * bm, 1, keepdims=True)
           + jnp.sum(cm * cm, 1, keepdims=True)) * inv
    r = lax.rsqrt(var + 1e-5)
    w1 = eW1[...]
    an = am * r * eg[:, :_D] + eb_[:, :_D]
    bn = bm * r * eg[:, _D:2 * _D] + eb_[:, _D:2 * _D]
    cn = cm * r * eg[:, 2 * _D:] + eb_[:, 2 * _D:]
    t = (jnp.dot(an, w1[:_D], preferred_element_type=jnp.float32)
         + jnp.dot(bn, w1[_D:2 * _D], preferred_element_type=jnp.float32)
         + jnp.dot(cn, w1[2 * _D:], preferred_element_type=jnp.float32)
         + eb1[...])
    t = jnp.maximum(t, 0.0)
    hen = cc + jnp.dot(t, eW2[...], preferred_element_type=jnp.float32) + eb2[...]
    he_out[...] = hen

    inv2 = 1.0 / (_D + _DE)
    mu2 = (jnp.sum(a, 1, keepdims=True) + jnp.sum(hen, 1, keepdims=True)) * inv2
    am2 = a - mu2
    cm2 = hen - mu2
    var2 = (jnp.sum(am2 * am2, 1, keepdims=True)
            + jnp.sum(cm2 * cm2, 1, keepdims=True)) * inv2
    r2 = lax.rsqrt(var2 + 1e-5)
    an2 = am2 * r2 * ng[:, :_D] + nb_[:, :_D]
    cn2 = cm2 * r2 * ng[:, _D:] + nb_[:, _D:]
    w1n = nW1[...]
    t2 = (jnp.dot(an2, w1n[:_D], preferred_element_type=jnp.float32)
          + jnp.dot(cn2, w1n[_D:], preferred_element_type=jnp.float32)
          + nb1[...])
    t2 = jnp.maximum(t2, 0.0)
    m_out[...] = jnp.dot(t2, nW2[...], preferred_element_type=jnp.float32) + nb2[...]


def _full(shape):
    return pl.BlockSpec(shape, lambda j: tuple(0 for _ in shape))


def _em_call(hs, hd, he, eW1, eb1, eW2, eb2, eg, eb_, nW1, nb1, nW2, nb2, ng, nb_):
    grid = _E // _BE
    espec = lambda w: pl.BlockSpec((_BE, w), lambda j: (j, 0))
    return pl.pallas_call(
        _em_kernel,
        grid=(grid,),
        in_specs=[
            espec(_D), espec(_D), espec(_DE),
            _full(eW1.shape), _full(eb1.shape), _full(eW2.shape), _full(eb2.shape),
            _full(eg.shape), _full(eb_.shape),
            _full(nW1.shape), _full(nb1.shape), _full(nW2.shape), _full(nb2.shape),
            _full(ng.shape), _full(nb_.shape),
        ],
        out_specs=[espec(_DE), espec(_D)],
        out_shape=[jax.ShapeDtypeStruct((_E, _DE), jnp.float32),
                   jax.ShapeDtypeStruct((_E, _D), jnp.float32)],
    )(hs, hd, he, eW1, eb1, eW2, eb2, eg, eb_, nW1, nb1, nW2, nb2, ng, nb_)


_BN = 2000


def _u_kernel(h, aggP, degP, Wu1, bu1, Wu2, bu2, u_out, deg_out):
    agg = aggP[0] + aggP[1]
    deg = jnp.maximum(degP[0] + degP[1], 1.0)
    aggn = agg / deg
    hh = h[...]
    w1 = Wu1[...]
    t = (jnp.dot(hh, w1[:_D], preferred_element_type=jnp.float32)
         + jnp.dot(aggn, w1[_D:], preferred_element_type=jnp.float32)
         + bu1[...])
    t = jnp.maximum(t, 0.0)
    u_out[...] = jnp.dot(t, Wu2[...], preferred_element_type=jnp.float32) + bu2[...]
    deg_out[...] = deg


def _u_call(h, aggP, degP, Wu1, bu1, Wu2, bu2):
    grid = _N // _BN
    nspec = pl.BlockSpec((_BN, _D), lambda j: (j, 0))
    pspec = pl.BlockSpec((_NC, _BN, _D), lambda j: (0, j, 0))
    return pl.pallas_call(
        _u_kernel,
        grid=(grid,),
        in_specs=[
            nspec, pspec, pspec,
            _full(Wu1.shape), _full(bu1.shape), _full(Wu2.shape), _full(bu2.shape),
        ],
        out_specs=[nspec, nspec],
        out_shape=[jax.ShapeDtypeStruct((_N, _D), jnp.float32),
                   jax.ShapeDtypeStruct((_N, _D), jnp.float32)],
    )(h, aggP, degP, Wu1, bu1, Wu2, bu2)


def _h_kernel(h, u, diffP, degb, out):
    dsum = diffP[0] + diffP[1]
    tau = jnp.tanh(dsum / degb[...])
    out[...] = (1.0 - tau) * h[...] + tau * u[...]


def _h_call(h, u, diffP, degb):
    grid = _N // _BN
    nspec = pl.BlockSpec((_BN, _D), lambda j: (j, 0))
    return pl.pallas_call(
        _h_kernel,
        grid=(grid,),
        in_specs=[nspec, nspec,
                  pl.BlockSpec((_NC, _BN, _D), lambda j: (0, j, 0)), nspec],
        out_specs=nspec,
        out_shape=jax.ShapeDtypeStruct((_N, _D), jnp.float32),
    )(h, u, diffP, degb)


def _f_kernel(h, pg, pb, lW, lb, mg, mb, loc_out, glob_out, acc):
    j = pl.program_id(0)
    hh = h[...]
    mu = jnp.mean(hh, 1, keepdims=True)
    hm = hh - mu
    var = jnp.mean(hm * hm, 1, keepdims=True)
    loc = hm * lax.rsqrt(var + 1e-5) * pg[...] + pb[...]
    loc_out[...] = loc

    @pl.when(j == 0)
    def _():
        acc[...] = jnp.zeros_like(acc)

    acc[...] += jnp.sum(loc, 0, keepdims=True)
    pooled = acc[...] * (1.0 / _N)
    g = jnp.dot(pooled, lW[...], preferred_element_type=jnp.float32) + lb[...]
    mu2 = jnp.mean(g, 1, keepdims=True)
    gm = g - mu2
    var2 = jnp.mean(gm * gm, 1, keepdims=True)
    glob_out[...] = gm * lax.rsqrt(var2 + 1e-5) * mg[...] + mb[...]


def _f_call(h, pg, pb, lW, lb, mg, mb):
    grid = _N // _BN
    return pl.pallas_call(
        _f_kernel,
        grid=(grid,),
        in_specs=[pl.BlockSpec((_BN, _D), lambda j: (j, 0)),
                  _full(pg.shape), _full(pb.shape), _full(lW.shape),
                  _full(lb.shape), _full(mg.shape), _full(mb.shape)],
        out_specs=[pl.BlockSpec((_BN, _D), lambda j: (j, 0)),
                   pl.BlockSpec((1, _D), lambda j: (0, 0))],
        out_shape=[jax.ShapeDtypeStruct((_N, _D), jnp.float32),
                   jax.ShapeDtypeStruct((1, _D), jnp.float32)],
        scratch_shapes=[pltpu.VMEM((1, _D), jnp.float32)],
    )(h, pg, pb, lW, lb, mg, mb)


def kernel(h, he, edge_index, eW1, eb1, eW2, eb2, eln_g, eln_b, nWm1, nbm1,
           nWm2, nbm2, nln_g, nln_b, nWu1, nbu1, nWu2, nbu2, pn_g, pn_b,
           lin_W, lin_b, mln_g, mln_b):
    src = edge_index[0]
    dst = edge_index[1]
    row = lambda v: v.reshape(1, -1)
    degP = None
    for i in (0, 1):
        if i == 0:
            hs, hd, degP = _sc_gather(True)(h, src, dst)
        else:
            hs, hd = _sc_gather(False)(h, src, dst)
        he, m = _em_call(hs, hd, he,
                         eW1[i], row(eb1[i]), eW2[i], row(eb2[i]),
                         row(eln_g[i]), row(eln_b[i]),
                         nWm1[i], row(nbm1[i]), nWm2[i], row(nbm2[i]),
                         row(nln_g[i]), row(nln_b[i]))
        aggP = _sc_segsum()(m, dst)
        u, degb = _u_call(h, aggP, degP, nWu1[i], row(nbu1[i]), nWu2[i], row(nbu2[i]))
        diffP = _sc_diff()(u, src, dst)
        h = _h_call(h, u, diffP, degb)
    return _f_call(h, row(pn_g), row(pn_b), lin_W, row(lin_b),
                   row(mln_g), row(mln_b))


# speedup vs baseline: 4.1663x; 1.2876x over previous
import functools

import jax
import jax.numpy as jnp
from jax import lax
from jax.experimental import pallas as pl
from jax.experimental.pallas import tpu as pltpu
from jax.experimental.pallas import tpu_sc as plsc

_N = 10000
_E = 160000
_D = 128
_DE = 16

_NC = 2
_NS = 16
_NW = _NC * _NS

_CH = 128
_EPW = _E // _NW
_NFULL = _EPW // _CH
_TAIL = _EPW - _NFULL * _CH
_CHG = 96
_NFG = _EPW // _CHG
_CHD = 56
_NFD = _EPW // _CHD
_TLD = _EPW - _NFD * _CHD


@functools.cache
def _sc_mesh():
    return plsc.VectorSubcoreMesh(
        core_axis_name="c", subcore_axis_name="s",
        num_cores=_NC, num_subcores=_NS)


_WB = 40
_NBLK = _N // _WB
_BLK_ROUNDS = (_NBLK + _NS - 1) // _NS


def _for_sub_blocks(s, fn):
    for kk in range(_BLK_ROUNDS):
        blk = s + _NS * kk

        @pl.when(blk < _NBLK)
        def _():
            fn(blk * _WB)


def _worker_id():
    c = lax.axis_index("c")
    s = lax.axis_index("s")
    return s * _NC + c, c, s


def _zero_vmem(ref, rows, cols):
    z = jnp.zeros((16,), jnp.float32)

    def body(r, cy):
        for j in range(cols // 16):
            ref[r, pl.ds(j * 16, 16)] = z
        return cy

    lax.fori_loop(0, rows, body, 0)


def _gather_pipeline(h_hbm, src_hbm, dst_hbm, hs_hbm, hd_hbm,
                     is_v, id_v, rs_v, rd_v, is8, id8, r8_v,
                     sem_is, sem_id, sem_gs, sem_gd, sem_ss, sem_sd, sem_t,
                     ch, nfull, on_dst_idx=None, on_dst_idx_tail=None):
    wid, _, _ = _worker_id()
    wb = wid * _EPW

    tb = wb + nfull * ch
    pltpu.sync_copy(src_hbm.at[pl.ds(tb, _TAIL)], is8)
    pltpu.sync_copy(dst_hbm.at[pl.ds(tb, _TAIL)], id8)
    pltpu.async_copy(h_hbm.at[is8], r8_v, sem_t).wait()
    pltpu.sync_copy(r8_v, hs_hbm.at[pl.ds(tb, _TAIL)])
    pltpu.async_copy(h_hbm.at[id8], r8_v, sem_t).wait()
    pltpu.sync_copy(r8_v, hd_hbm.at[pl.ds(tb, _TAIL)])
    if on_dst_idx_tail is not None:
        on_dst_idx_tail(id8)

    def istart(j):
        b = wb + j * ch
        pltpu.async_copy(src_hbm.at[pl.ds(b, ch)], is_v, sem_is)
        pltpu.async_copy(dst_hbm.at[pl.ds(b, ch)], id_v, sem_id)

    def iwait(j):
        b = wb + j * ch
        pltpu.make_async_copy(src_hbm.at[pl.ds(b, ch)], is_v, sem_is).wait()
        pltpu.make_async_copy(dst_hbm.at[pl.ds(b, ch)], id_v, sem_id).wait()

    def swait(j):
        b = wb + j * ch
        pltpu.make_async_copy(rs_v, hs_hbm.at[pl.ds(b, ch)], sem_ss).wait()
        pltpu.make_async_copy(rd_v, hd_hbm.at[pl.ds(b, ch)], sem_sd).wait()

    istart(0)

    def body(j, cy):
        b = wb + j * ch
        iwait(j)
        if on_dst_idx is not None:
            on_dst_idx(id_v)

        @pl.when(j > 0)
        def _():
            swait(j - 1)

        g1 = pltpu.async_copy(h_hbm.at[is_v], rs_v, sem_gs)
        g2 = pltpu.async_copy(h_hbm.at[id_v], rd_v, sem_gd)
        g1.wait()
        g2.wait()

        @pl.when(j < nfull - 1)
        def _():
            istart(j + 1)

        pltpu.async_copy(rs_v, hs_hbm.at[pl.ds(b, ch)], sem_ss)
        pltpu.async_copy(rd_v, hd_hbm.at[pl.ds(b, ch)], sem_sd)
        return cy

    lax.fori_loop(0, nfull, body, 0)
    swait(nfull - 1)


def _sc_gather_deg_body(h_hbm, src_hbm, dst_hbm, hs_hbm, hd_hbm, deg_hbm,
                        acc_sh, is_v, id_v, rs_v, rd_v, is8, id8, r8_v, ones_v,
                        zb_v, sem_is, sem_id, sem_gs, sem_gd, sem_ss, sem_sd,
                        sem_t):
    wid, c, s = _worker_id()
    _zero_vmem(zb_v, _WB, _D)
    _for_sub_blocks(s, lambda off: pltpu.sync_copy(zb_v, acc_sh.at[pl.ds(off, _WB)]))

    def fill_ones(r, cy):
        one = jnp.ones((16,), jnp.float32)
        for j in range(_D // 16):
            ones_v[r, pl.ds(j * 16, 16)] = one
        return cy

    lax.fori_loop(0, _CHG, fill_ones, 0)
    plsc.subcore_barrier()

    _gather_pipeline(
        h_hbm, src_hbm, dst_hbm, hs_hbm, hd_hbm,
        is_v, id_v, rs_v, rd_v, is8, id8, r8_v,
        sem_is, sem_id, sem_gs, sem_gd, sem_ss, sem_sd, sem_t,
        _CHG, _NFG,
        on_dst_idx=lambda idx: pltpu.sync_copy(ones_v, acc_sh.at[idx], add=True),
        on_dst_idx_tail=lambda idx: pltpu.sync_copy(
            ones_v.at[pl.ds(0, _TAIL)], acc_sh.at[idx], add=True),
    )
    plsc.subcore_barrier()

    def wbk(off):
        pltpu.sync_copy(acc_sh.at[pl.ds(off, _WB)], zb_v)
        pltpu.sync_copy(zb_v, deg_hbm.at[c, pl.ds(off, _WB)])

    _for_sub_blocks(s, wbk)


def _sc_gather_body(h_hbm, src_hbm, dst_hbm, hs_hbm, hd_hbm,
                    is_v, id_v, rs_v, rd_v, is8, id8, r8_v,
                    sem_is, sem_id, sem_gs, sem_gd, sem_ss, sem_sd, sem_t):
    _gather_pipeline(h_hbm, src_hbm, dst_hbm, hs_hbm, hd_hbm,
                     is_v, id_v, rs_v, rd_v, is8, id8, r8_v,
                     sem_is, sem_id, sem_gs, sem_gd, sem_ss, sem_sd, sem_t,
                     _CH, _NFULL)


_GATHER_SCRATCH = [
    pltpu.VMEM((_CH,), jnp.int32),
    pltpu.VMEM((_CH,), jnp.int32),
    pltpu.VMEM((_CH, _D), jnp.float32),
    pltpu.VMEM((_CH, _D), jnp.float32),
    pltpu.VMEM((_TAIL,), jnp.int32),
    pltpu.VMEM((_TAIL,), jnp.int32),
    pltpu.VMEM((_TAIL, _D), jnp.float32),
] + [pltpu.SemaphoreType.DMA] * 7


@functools.cache
def _sc_gather(with_deg):
    if with_deg:
        return pl.kernel(
            _sc_gather_deg_body,
            out_type=(jax.ShapeDtypeStruct((_E, _D), jnp.float32),
                      jax.ShapeDtypeStruct((_E, _D), jnp.float32),
                      jax.ShapeDtypeStruct((_NC, _N, _D), jnp.float32)),
            mesh=_sc_mesh(),
            scratch_types=(
                [pltpu.VMEM_SHARED((_N, _D), jnp.float32),
                 pltpu.VMEM((_CHG,), jnp.int32),
                 pltpu.VMEM((_CHG,), jnp.int32),
                 pltpu.VMEM((_CHG, _D), jnp.float32),
                 pltpu.VMEM((_CHG, _D), jnp.float32),
                 pltpu.VMEM((_TAIL,), jnp.int32),
                 pltpu.VMEM((_TAIL,), jnp.int32),
                 pltpu.VMEM((_TAIL, _D), jnp.float32),
                 pltpu.VMEM((_CHG, _D), jnp.float32),
                 pltpu.VMEM((_WB, _D), jnp.float32)]
                + [pltpu.SemaphoreType.DMA] * 7
            ),
        )
    return pl.kernel(
        _sc_gather_body,
        out_type=(jax.ShapeDtypeStruct((_E, _D), jnp.float32),
                  jax.ShapeDtypeStruct((_E, _D), jnp.float32)),
        mesh=_sc_mesh(),
        scratch_types=list(_GATHER_SCRATCH),
    )


def _sc_segsum_body(vals_hbm, dst_hbm, out_hbm, acc_sh,
                    rows_a, rows_b, idx_a, idx_b, rows8, idx8, zb_v,
                    sem_ra, sem_rb, sem_ia, sem_ib):
    wid, c, s = _worker_id()
    wbase = wid * _EPW
    _zero_vmem(zb_v, _WB, _D)
    _for_sub_blocks(s, lambda off: pltpu.sync_copy(zb_v, acc_sh.at[pl.ds(off, _WB)]))
    plsc.subcore_barrier()

    bufs = ((rows_a, idx_a, sem_ra, sem_ia), (rows_b, idx_b, sem_rb, sem_ib))

    def lstart(j, p):
        rv, iv, sr, si = bufs[p]
        b = wbase + j * _CH
        pltpu.async_copy(vals_hbm.at[pl.ds(b, _CH)], rv, sr)
        pltpu.async_copy(dst_hbm.at[pl.ds(b, _CH)], iv, si)

    def lwait(j, p):
        rv, iv, sr, si = bufs[p]
        b = wbase + j * _CH
        pltpu.make_async_copy(vals_hbm.at[pl.ds(b, _CH)], rv, sr).wait()
        pltpu.make_async_copy(dst_hbm.at[pl.ds(b, _CH)], iv, si).wait()

    def scatter(p):
        rv, iv, _, _ = bufs[p]
        pltpu.sync_copy(rv, acc_sh.at[iv], add=True)

    lstart(0, 0)

    def body(k, cy):
        lwait(2 * k, 0)
        lstart(2 * k + 1, 1)
        scatter(0)
        lwait(2 * k + 1, 1)
        lstart(2 * k + 2, 0)
        scatter(1)
        return cy

    lax.fori_loop(0, (_NFULL - 1) // 2, body, 0)
    lwait(_NFULL - 1, 0)
    tb = wbase + _NFULL * _CH
    pltpu.sync_copy(vals_hbm.at[pl.ds(tb, _TAIL)], rows8)
    pltpu.sync_copy(dst_hbm.at[pl.ds(tb, _TAIL)], idx8)
    scatter(0)
    pltpu.sync_copy(rows8, acc_sh.at[idx8], add=True)
    plsc.subcore_barrier()

    def wb(off):
        pltpu.sync_copy(acc_sh.at[pl.ds(off, _WB)], zb_v)
        pltpu.sync_copy(zb_v, out_hbm.at[c, pl.ds(off, _WB)])

    _for_sub_blocks(s, wb)


@functools.cache
def _sc_segsum():
    return pl.kernel(
        _sc_segsum_body,
        out_type=jax.ShapeDtypeStruct((_NC, _N, _D), jnp.float32),
        mesh=_sc_mesh(),
        scratch_types=[
            pltpu.VMEM_SHARED((_N, _D), jnp.float32),
            pltpu.VMEM((_CH, _D), jnp.float32),
            pltpu.VMEM((_CH, _D), jnp.float32),
            pltpu.VMEM((_CH,), jnp.int32),
            pltpu.VMEM((_CH,), jnp.int32),
            pltpu.VMEM((_TAIL, _D), jnp.float32),
            pltpu.VMEM((_TAIL,), jnp.int32),
            pltpu.VMEM((_WB, _D), jnp.float32),
        ] + [pltpu.SemaphoreType.DMA] * 4,
    )


def _sq_rows(us, ud, nrows):
    def row(r, rcy):
        for jj in range(_D // 16):
            sl = pl.ds(jj * 16, 16)
            d = us[r, sl] - ud[r, sl]
            us[r, sl] = d * d
        return rcy

    lax.fori_loop(0, nrows, row, 0)


def _sc_diff_body(u_hbm, src_hbm, dst_hbm, out_hbm, acc_sh,
                  si_a, di_a, si_b, di_b, us_a, ud_a, us_b, ud_b,
                  si8, di8, us8, ud8, zb_v,
                  sem_ia, sem_ib, sem_ja, sem_jb,
                  sem_ga, sem_gb, sem_ha, sem_hb, sem_t):
    wid, c, s = _worker_id()
    wbase = wid * _EPW
    _zero_vmem(zb_v, _WB, _D)
    _for_sub_blocks(s, lambda off: pltpu.sync_copy(zb_v, acc_sh.at[pl.ds(off, _WB)]))
    plsc.subcore_barrier()

    bufs = ((si_a, di_a, us_a, ud_a, sem_ia, sem_ja, sem_ga, sem_ha),
            (si_b, di_b, us_b, ud_b, sem_ib, sem_jb, sem_gb, sem_hb))

    def istart(j, p):
        si, di, _, _, s_i, s_j, _, _ = bufs[p]
        b = wbase + j * _CHD
        pltpu.async_copy(src_hbm.at[pl.ds(b, _CHD)], si, s_i)
        pltpu.async_copy(dst_hbm.at[pl.ds(b, _CHD)], di, s_j)

    def iwait(j, p):
        si, di, _, _, s_i, s_j, _, _ = bufs[p]
        b = wbase + j * _CHD
        pltpu.make_async_copy(src_hbm.at[pl.ds(b, _CHD)], si, s_i).wait()
        pltpu.make_async_copy(dst_hbm.at[pl.ds(b, _CHD)], di, s_j).wait()

    def gstart(p):
        si, di, us, ud, _, _, s_g, s_h = bufs[p]
        pltpu.async_copy(u_hbm.at[si], us, s_g)
        pltpu.async_copy(u_hbm.at[di], ud, s_h)

    def gwait(p):
        si, di, us, ud, _, _, s_g, s_h = bufs[p]
        pltpu.make_async_copy(u_hbm.at[si], us, s_g).wait()
        pltpu.make_async_copy(u_hbm.at[di], ud, s_h).wait()

    def comp_scat(p):
        _, di, us, ud, _, _, _, _ = bufs[p]
        _sq_rows(us, ud, _CHD)
        pltpu.sync_copy(us, acc_sh.at[di], add=True)

    def half(j, x, y, next_idx):
        iwait(j + 1, y)
        gstart(y)
        gwait(x)
        comp_scat(x)
        if next_idx:
            istart(j + 2, x)

    istart(0, 0)
    iwait(0, 0)
    gstart(0)
    istart(1, 1)

    def body(k, cy):
        half(2 * k, 0, 1, True)
        half(2 * k + 1, 1, 0, True)
        return cy

    lax.fori_loop(0, (_NFD - 3) // 2, body, 0)
    half(_NFD - 3, 0, 1, True)
    half(_NFD - 2, 1, 0, False)
    gwait(0)
    tb = wbase + _NFD * _CHD
    pltpu.sync_copy(src_hbm.at[pl.ds(tb, _TLD)], si8)
    pltpu.sync_copy(dst_hbm.at[pl.ds(tb, _TLD)], di8)
    t1 = pltpu.async_copy(u_hbm.at[si8], us8, sem_t)
    t2 = pltpu.async_copy(u_hbm.at[di8], ud8, sem_t)
    comp_scat(0)
    t1.wait()
    t2.wait()
    _sq_rows(us8, ud8, _TLD)
    pltpu.sync_copy(us8, acc_sh.at[di8], add=True)
    plsc.subcore_barrier()

    def wb(off):
        pltpu.sync_copy(acc_sh.at[pl.ds(off, _WB)], zb_v)
        pltpu.sync_copy(zb_v, out_hbm.at[c, pl.ds(off, _WB)])

    _for_sub_blocks(s, wb)


@functools.cache
def _sc_diff():
    return pl.kernel(
        _sc_diff_body,
        out_type=jax.ShapeDtypeStruct((_NC, _N, _D), jnp.float32),
        mesh=_sc_mesh(),
        scratch_types=[
            pltpu.VMEM_SHARED((_N, _D), jnp.float32),
            pltpu.VMEM((_CHD,), jnp.int32),
            pltpu.VMEM((_CHD,), jnp.int32),
            pltpu.VMEM((_CHD,), jnp.int32),
            pltpu.VMEM((_CHD,), jnp.int32),
            pltpu.VMEM((_CHD, _D), jnp.float32),
            pltpu.VMEM((_CHD, _D), jnp.float32),
            pltpu.VMEM((_CHD, _D), jnp.float32),
            pltpu.VMEM((_CHD, _D), jnp.float32),
            pltpu.VMEM((_TLD,), jnp.int32),
            pltpu.VMEM((_TLD,), jnp.int32),
            pltpu.VMEM((_TLD, _D), jnp.float32),
            pltpu.VMEM((_TLD, _D), jnp.float32),
            pltpu.VMEM((_WB, _D), jnp.float32),
        ] + [pltpu.SemaphoreType.DMA] * 9,
    )


_BE = 2000


def _em_kernel(hs, hd, he, eW1, eb1, eW2, eb2, eg, eb_, nW1, nb1, nW2, nb2,
               ng, nb_, he_out, m_out):
    a = hs[...]
    b = hd[...]
    cc = he[...]
    inv = 1.0 / (2 * _D + _DE)
    mu = (jnp.sum(a, 1, keepdims=True) + jnp.sum(b, 1, keepdims=True)
          + jnp.sum(cc, 1, keepdims=True)) * inv
    am = a - mu
    bm = b - mu
    cm = cc - mu
    var = (jnp.sum(am * am, 1, keepdims=True) + jnp.sum(bm * bm, 1, keepdims=True)
           + jnp.sum(cm * cm, 1, keepdims=True)) * inv
    r = lax.rsqrt(var + 1e-5)
    w1 = eW1[...]
    an = am * r * eg[:, :_D] + eb_[:, :_D]
    bn = bm * r * eg[:, _D:2 * _D] + eb_[:, _D:2 * _D]
    cn = cm * r * eg[:, 2 * _D:] + eb_[:, 2 * _D:]
    t = (jnp.dot(an, w1[:_D], preferred_element_type=jnp.float32)
         + jnp.dot(bn, w1[_D:2 * _D], preferred_element_type=jnp.float32)
         + jnp.dot(cn, w1[2 * _D:], preferred_element_type=jnp.float32)
         + eb1[...])
    t = jnp.maximum(t, 0.0)
    hen = cc + jnp.dot(t, eW2[...], preferred_element_type=jnp.float32) + eb2[...]
    he_out[...] = hen

    inv2 = 1.0 / (_D + _DE)
    mu2 = (jnp.sum(a, 1, keepdims=True) + jnp.sum(hen, 1, keepdims=True)) * inv2
    am2 = a - mu2
    cm2 = hen - mu2
    var2 = (jnp.sum(am2 * am2, 1, keepdims=True)
            + jnp.sum(cm2 * cm2, 1, keepdims=True)) * inv2
    r2 = lax.rsqrt(var2 + 1e-5)
    an2 = am2 * r2 * ng[:, :_D] + nb_[:, :_D]
    cn2 = cm2 * r2 * ng[:, _D:] + nb_[:, _D:]
    w1n = nW1[...]
    t2 = (jnp.dot(an2, w1n[:_D], preferred_element_type=jnp.float32)
          + jnp.dot(cn2, w1n[_D:], preferred_element_type=jnp.float32)
          + nb1[...])
    t2 = jnp.maximum(t2, 0.0)
    m_out[...] = jnp.dot(t2, nW2[...], preferred_element_type=jnp.float32) + nb2[...]


def _full(shape):
    return pl.BlockSpec(shape, lambda j: tuple(0 for _ in shape))


def _em_call(hs, hd, he, eW1, eb1, eW2, eb2, eg, eb_, nW1, nb1, nW2, nb2, ng, nb_):
    grid = _E // _BE
    espec = lambda w: pl.BlockSpec((_BE, w), lambda j: (j, 0))
    return pl.pallas_call(
        _em_kernel,
        grid=(grid,),
        in_specs=[
            espec(_D), espec(_D), espec(_DE),
            _full(eW1.shape), _full(eb1.shape), _full(eW2.shape), _full(eb2.shape),
            _full(eg.shape), _full(eb_.shape),
            _full(nW1.shape), _full(nb1.shape), _full(nW2.shape), _full(nb2.shape),
            _full(ng.shape), _full(nb_.shape),
        ],
        out_specs=[espec(_DE), espec(_D)],
        out_shape=[jax.ShapeDtypeStruct((_E, _DE), jnp.float32),
                   jax.ShapeDtypeStruct((_E, _D), jnp.float32)],
    )(hs, hd, he, eW1, eb1, eW2, eb2, eg, eb_, nW1, nb1, nW2, nb2, ng, nb_)


_BN = 2000


def _u_kernel(h, aggP, degP, Wu1, bu1, Wu2, bu2, u_out, deg_out):
    agg = aggP[0] + aggP[1]
    deg = jnp.maximum(degP[0] + degP[1], 1.0)
    aggn = agg / deg
    hh = h[...]
    w1 = Wu1[...]
    t = (jnp.dot(hh, w1[:_D], preferred_element_type=jnp.float32)
         + jnp.dot(aggn, w1[_D:], preferred_element_type=jnp.float32)
         + bu1[...])
    t = jnp.maximum(t, 0.0)
    u_out[...] = jnp.dot(t, Wu2[...], preferred_element_type=jnp.float32) + bu2[...]
    deg_out[...] = deg


def _u_call(h, aggP, degP, Wu1, bu1, Wu2, bu2):
    grid = _N // _BN
    nspec = pl.BlockSpec((_BN, _D), lambda j: (j, 0))
    pspec = pl.BlockSpec((_NC, _BN, _D), lambda j: (0, j, 0))
    return pl.pallas_call(
        _u_kernel,
        grid=(grid,),
        in_specs=[
            nspec, pspec, pspec,
            _full(Wu1.shape), _full(bu1.shape), _full(Wu2.shape), _full(bu2.shape),
        ],
        out_specs=[nspec, nspec],
        out_shape=[jax.ShapeDtypeStruct((_N, _D), jnp.float32),
                   jax.ShapeDtypeStruct((_N, _D), jnp.float32)],
    )(h, aggP, degP, Wu1, bu1, Wu2, bu2)


def _h_kernel(h, u, diffP, degb, out):
    dsum = diffP[0] + diffP[1]
    tau = jnp.tanh(dsum / degb[...])
    out[...] = (1.0 - tau) * h[...] + tau * u[...]


def _h_call(h, u, diffP, degb):
    grid = _N // _BN
    nspec = pl.BlockSpec((_BN, _D), lambda j: (j, 0))
    return pl.pallas_call(
        _h_kernel,
        grid=(grid,),
        in_specs=[nspec, nspec,
                  pl.BlockSpec((_NC, _BN, _D), lambda j: (0, j, 0)), nspec],
        out_specs=nspec,
        out_shape=jax.ShapeDtypeStruct((_N, _D), jnp.float32),
    )(h, u, diffP, degb)


def _f_kernel(h, pg, pb, lW, lb, mg, mb, loc_out, glob_out, acc):
    j = pl.program_id(0)
    hh = h[...]
    mu = jnp.mean(hh, 1, keepdims=True)
    hm = hh - mu
    var = jnp.mean(hm * hm, 1, keepdims=True)
    loc = hm * lax.rsqrt(var + 1e-5) * pg[...] + pb[...]
    loc_out[...] = loc

    @pl.when(j == 0)
    def _():
        acc[...] = jnp.zeros_like(acc)

    acc[...] += jnp.sum(loc, 0, keepdims=True)
    pooled = acc[...] * (1.0 / _N)
    g = jnp.dot(pooled, lW[...], preferred_element_type=jnp.float32) + lb[...]
    mu2 = jnp.mean(g, 1, keepdims=True)
    gm = g - mu2
    var2 = jnp.mean(gm * gm, 1, keepdims=True)
    glob_out[...] = gm * lax.rsqrt(var2 + 1e-5) * mg[...] + mb[...]


def _f_call(h, pg, pb, lW, lb, mg, mb):
    grid = _N // _BN
    return pl.pallas_call(
        _f_kernel,
        grid=(grid,),
        in_specs=[pl.BlockSpec((_BN, _D), lambda j: (j, 0)),
                  _full(pg.shape), _full(pb.shape), _full(lW.shape),
                  _full(lb.shape), _full(mg.shape), _full(mb.shape)],
        out_specs=[pl.BlockSpec((_BN, _D), lambda j: (j, 0)),
                   pl.BlockSpec((1, _D), lambda j: (0, 0))],
        out_shape=[jax.ShapeDtypeStruct((_N, _D), jnp.float32),
                   jax.ShapeDtypeStruct((1, _D), jnp.float32)],
        scratch_shapes=[pltpu.VMEM((1, _D), jnp.float32)],
    )(h, pg, pb, lW, lb, mg, mb)


def kernel(h, he, edge_index, eW1, eb1, eW2, eb2, eln_g, eln_b, nWm1, nbm1,
           nWm2, nbm2, nln_g, nln_b, nWu1, nbu1, nWu2, nbu2, pn_g, pn_b,
           lin_W, lin_b, mln_g, mln_b):
    src = edge_index[0]
    dst = edge_index[1]
    row = lambda v: v.reshape(1, -1)
    degP = None
    for i in (0, 1):
        if i == 0:
            hs, hd, degP = _sc_gather(True)(h, src, dst)
        else:
            hs, hd = _sc_gather(False)(h, src, dst)
        he, m = _em_call(hs, hd, he,
                         eW1[i], row(eb1[i]), eW2[i], row(eb2[i]),
                         row(eln_g[i]), row(eln_b[i]),
                         nWm1[i], row(nbm1[i]), nWm2[i], row(nbm2[i]),
                         row(nln_g[i]), row(nln_b[i]))
        aggP = _sc_segsum()(m, dst)
        u, degb = _u_call(h, aggP, degP, nWu1[i], row(nbu1[i]), nWu2[i], row(nbu2[i]))
        diffP = _sc_diff()(u, src, dst)
        h = _h_call(h, u, diffP, degb)
    return _f_call(h, row(pn_g), row(pn_b), lin_W, row(lin_b),
                   row(mln_g), row(mln_b))


# speedup vs baseline: 4.2986x; 1.0318x over previous
import functools

import jax
import jax.numpy as jnp
from jax import lax
from jax.experimental import pallas as pl
from jax.experimental.pallas import tpu as pltpu
from jax.experimental.pallas import tpu_sc as plsc

_N = 10000
_E = 160000
_D = 128
_DE = 16

_NC = 2
_NS = 16
_NW = _NC * _NS

_CH = 128
_EPW = _E // _NW
_NFULL = _EPW // _CH
_TAIL = _EPW - _NFULL * _CH
_CHG = 96
_NFG = _EPW // _CHG
_CHD = 56
_NFD = _EPW // _CHD
_TLD = _EPW - _NFD * _CHD


@functools.cache
def _sc_mesh():
    return plsc.VectorSubcoreMesh(
        core_axis_name="c", subcore_axis_name="s",
        num_cores=_NC, num_subcores=_NS)


_WB = 40
_NBLK = _N // _WB
_BLK_ROUNDS = (_NBLK + _NS - 1) // _NS


def _for_sub_blocks(s, fn):
    for kk in range(_BLK_ROUNDS):
        blk = s + _NS * kk

        @pl.when(blk < _NBLK)
        def _():
            fn(blk * _WB)


def _worker_id():
    c = lax.axis_index("c")
    s = lax.axis_index("s")
    return s * _NC + c, c, s


def _zero_vmem(ref, rows, cols):
    z = jnp.zeros((16,), jnp.float32)

    def body(r, cy):
        for j in range(cols // 16):
            ref[r, pl.ds(j * 16, 16)] = z
        return cy

    lax.fori_loop(0, rows, body, 0)


def _gather_pipeline(h_hbm, src_hbm, dst_hbm, hs_hbm, hd_hbm,
                     is_v, id_v, rs_v, rd_v, is8, id8, r8_v,
                     sem_is, sem_id, sem_gs, sem_gd, sem_ss, sem_sd, sem_t,
                     ch, nfull, on_dst_idx=None, on_dst_idx_tail=None):
    wid, _, _ = _worker_id()
    wb = wid * _EPW

    tb = wb + nfull * ch
    pltpu.sync_copy(src_hbm.at[pl.ds(tb, _TAIL)], is8)
    pltpu.sync_copy(dst_hbm.at[pl.ds(tb, _TAIL)], id8)
    pltpu.async_copy(h_hbm.at[is8], r8_v, sem_t).wait()
    pltpu.sync_copy(r8_v, hs_hbm.at[pl.ds(tb, _TAIL)])
    pltpu.async_copy(h_hbm.at[id8], r8_v, sem_t).wait()
    pltpu.sync_copy(r8_v, hd_hbm.at[pl.ds(tb, _TAIL)])
    if on_dst_idx_tail is not None:
        on_dst_idx_tail(id8)

    def istart(j):
        b = wb + j * ch
        pltpu.async_copy(src_hbm.at[pl.ds(b, ch)], is_v, sem_is)
        pltpu.async_copy(dst_hbm.at[pl.ds(b, ch)], id_v, sem_id)

    def iwait(j):
        b = wb + j * ch
        pltpu.make_async_copy(src_hbm.at[pl.ds(b, ch)], is_v, sem_is).wait()
        pltpu.make_async_copy(dst_hbm.at[pl.ds(b, ch)], id_v, sem_id).wait()

    def swait(j):
        b = wb + j * ch
        pltpu.make_async_copy(rs_v, hs_hbm.at[pl.ds(b, ch)], sem_ss).wait()
        pltpu.make_async_copy(rd_v, hd_hbm.at[pl.ds(b, ch)], sem_sd).wait()

    istart(0)

    def body(j, cy):
        b = wb + j * ch
        iwait(j)
        if on_dst_idx is not None:
            on_dst_idx(id_v)

        @pl.when(j > 0)
        def _():
            swait(j - 1)

        g1 = pltpu.async_copy(h_hbm.at[is_v], rs_v, sem_gs)
        g2 = pltpu.async_copy(h_hbm.at[id_v], rd_v, sem_gd)
        g1.wait()
        g2.wait()

        @pl.when(j < nfull - 1)
        def _():
            istart(j + 1)

        pltpu.async_copy(rs_v, hs_hbm.at[pl.ds(b, ch)], sem_ss)
        pltpu.async_copy(rd_v, hd_hbm.at[pl.ds(b, ch)], sem_sd)
        return cy

    lax.fori_loop(0, nfull, body, 0)
    swait(nfull - 1)


def _sc_gather_deg_body(h_hbm, src_hbm, dst_hbm, hs_hbm, hd_hbm, deg_hbm,
                        acc_sh, is_v, id_v, rs_v, rd_v, is8, id8, r8_v, ones_v,
                        zb_v, sem_is, sem_id, sem_gs, sem_gd, sem_ss, sem_sd,
                        sem_t):
    wid, c, s = _worker_id()
    _zero_vmem(zb_v, _WB, _D)
    _for_sub_blocks(s, lambda off: pltpu.sync_copy(zb_v, acc_sh.at[pl.ds(off, _WB)]))

    def fill_ones(r, cy):
        one = jnp.ones((16,), jnp.float32)
        for j in range(_D // 16):
            ones_v[r, pl.ds(j * 16, 16)] = one
        return cy

    lax.fori_loop(0, _CHG, fill_ones, 0)
    plsc.subcore_barrier()

    _gather_pipeline(
        h_hbm, src_hbm, dst_hbm, hs_hbm, hd_hbm,
        is_v, id_v, rs_v, rd_v, is8, id8, r8_v,
        sem_is, sem_id, sem_gs, sem_gd, sem_ss, sem_sd, sem_t,
        _CHG, _NFG,
        on_dst_idx=lambda idx: pltpu.sync_copy(ones_v, acc_sh.at[idx], add=True),
        on_dst_idx_tail=lambda idx: pltpu.sync_copy(
            ones_v.at[pl.ds(0, _TAIL)], acc_sh.at[idx], add=True),
    )
    plsc.subcore_barrier()

    def wbk(off):
        pltpu.sync_copy(acc_sh.at[pl.ds(off, _WB)], zb_v)
        pltpu.sync_copy(zb_v, deg_hbm.at[c, pl.ds(off, _WB)])

    _for_sub_blocks(s, wbk)


def _sc_gather_body(h_hbm, src_hbm, dst_hbm, hs_hbm, hd_hbm,
                    is_v, id_v, rs_v, rd_v, is8, id8, r8_v,
                    sem_is, sem_id, sem_gs, sem_gd, sem_ss, sem_sd, sem_t):
    _gather_pipeline(h_hbm, src_hbm, dst_hbm, hs_hbm, hd_hbm,
                     is_v, id_v, rs_v, rd_v, is8, id8, r8_v,
                     sem_is, sem_id, sem_gs, sem_gd, sem_ss, sem_sd, sem_t,
                     _CH, _NFULL)


_GATHER_SCRATCH = [
    pltpu.VMEM((_CH,), jnp.int32),
    pltpu.VMEM((_CH,), jnp.int32),
    pltpu.VMEM((_CH, _D), jnp.float32),
    pltpu.VMEM((_CH, _D), jnp.float32),
    pltpu.VMEM((_TAIL,), jnp.int32),
    pltpu.VMEM((_TAIL,), jnp.int32),
    pltpu.VMEM((_TAIL, _D), jnp.float32),
] + [pltpu.SemaphoreType.DMA] * 7


@functools.cache
def _sc_gather(with_deg):
    if with_deg:
        return pl.kernel(
            _sc_gather_deg_body,
            out_type=(jax.ShapeDtypeStruct((_E, _D), jnp.float32),
                      jax.ShapeDtypeStruct((_E, _D), jnp.float32),
                      jax.ShapeDtypeStruct((_NC, _N, _D), jnp.float32)),
            mesh=_sc_mesh(),
            scratch_types=(
                [pltpu.VMEM_SHARED((_N, _D), jnp.float32),
                 pltpu.VMEM((_CHG,), jnp.int32),
                 pltpu.VMEM((_CHG,), jnp.int32),
                 pltpu.VMEM((_CHG, _D), jnp.float32),
                 pltpu.VMEM((_CHG, _D), jnp.float32),
                 pltpu.VMEM((_TAIL,), jnp.int32),
                 pltpu.VMEM((_TAIL,), jnp.int32),
                 pltpu.VMEM((_TAIL, _D), jnp.float32),
                 pltpu.VMEM((_CHG, _D), jnp.float32),
                 pltpu.VMEM((_WB, _D), jnp.float32)]
                + [pltpu.SemaphoreType.DMA] * 7
            ),
        )
    return pl.kernel(
        _sc_gather_body,
        out_type=(jax.ShapeDtypeStruct((_E, _D), jnp.float32),
                  jax.ShapeDtypeStruct((_E, _D), jnp.float32)),
        mesh=_sc_mesh(),
        scratch_types=list(_GATHER_SCRATCH),
    )


def _sc_segsum_body(vals_hbm, dst_hbm, out_hbm, acc_sh,
                    rows_a, rows_b, idx_a, idx_b, rows8, idx8, zb_v,
                    sem_ra, sem_rb, sem_ia, sem_ib):
    wid, c, s = _worker_id()
    wbase = wid * _EPW
    _zero_vmem(zb_v, _WB, _D)
    _for_sub_blocks(s, lambda off: pltpu.sync_copy(zb_v, acc_sh.at[pl.ds(off, _WB)]))
    plsc.subcore_barrier()

    bufs = ((rows_a, idx_a, sem_ra, sem_ia), (rows_b, idx_b, sem_rb, sem_ib))

    def lstart(j, p):
        rv, iv, sr, si = bufs[p]
        b = wbase + j * _CH
        pltpu.async_copy(vals_hbm.at[pl.ds(b, _CH)], rv, sr)
        pltpu.async_copy(dst_hbm.at[pl.ds(b, _CH)], iv, si)

    def lwait(j, p):
        rv, iv, sr, si = bufs[p]
        b = wbase + j * _CH
        pltpu.make_async_copy(vals_hbm.at[pl.ds(b, _CH)], rv, sr).wait()
        pltpu.make_async_copy(dst_hbm.at[pl.ds(b, _CH)], iv, si).wait()

    def scatter(p):
        rv, iv, _, _ = bufs[p]
        pltpu.sync_copy(rv, acc_sh.at[iv], add=True)

    lstart(0, 0)

    def body(k, cy):
        lwait(2 * k, 0)
        lstart(2 * k + 1, 1)
        scatter(0)
        lwait(2 * k + 1, 1)
        lstart(2 * k + 2, 0)
        scatter(1)
        return cy

    lax.fori_loop(0, (_NFULL - 1) // 2, body, 0)
    lwait(_NFULL - 1, 0)
    tb = wbase + _NFULL * _CH
    pltpu.sync_copy(vals_hbm.at[pl.ds(tb, _TAIL)], rows8)
    pltpu.sync_copy(dst_hbm.at[pl.ds(tb, _TAIL)], idx8)
    scatter(0)
    pltpu.sync_copy(rows8, acc_sh.at[idx8], add=True)
    plsc.subcore_barrier()

    def wb(off):
        pltpu.sync_copy(acc_sh.at[pl.ds(off, _WB)], zb_v)
        pltpu.sync_copy(zb_v, out_hbm.at[c, pl.ds(off, _WB)])

    _for_sub_blocks(s, wb)


@functools.cache
def _sc_segsum():
    return pl.kernel(
        _sc_segsum_body,
        out_type=jax.ShapeDtypeStruct((_NC, _N, _D), jnp.float32),
        mesh=_sc_mesh(),
        scratch_types=[
            pltpu.VMEM_SHARED((_N, _D), jnp.float32),
            pltpu.VMEM((_CH, _D), jnp.float32),
            pltpu.VMEM((_CH, _D), jnp.float32),
            pltpu.VMEM((_CH,), jnp.int32),
            pltpu.VMEM((_CH,), jnp.int32),
            pltpu.VMEM((_TAIL, _D), jnp.float32),
            pltpu.VMEM((_TAIL,), jnp.int32),
            pltpu.VMEM((_WB, _D), jnp.float32),
        ] + [pltpu.SemaphoreType.DMA] * 4,
    )


def _sq_rows(us, ud, nrows):
    def row(r, rcy):
        for jj in range(_D // 16):
            sl = pl.ds(jj * 16, 16)
            d = us[r, sl] - ud[r, sl]
            us[r, sl] = d * d
        return rcy

    lax.fori_loop(0, nrows, row, 0)


def _sc_diff_body(u_hbm, src_hbm, dst_hbm, out_hbm, acc_sh,
                  si_a, di_a, si_b, di_b, us_a, ud_a, us_b, ud_b,
                  si8, di8, us8, ud8, zb_v,
                  sem_ia, sem_ib, sem_ja, sem_jb,
                  sem_ga, sem_gb, sem_ha, sem_hb, sem_t):
    wid, c, s = _worker_id()
    wbase = wid * _EPW
    _zero_vmem(zb_v, _WB, _D)
    _for_sub_blocks(s, lambda off: pltpu.sync_copy(zb_v, acc_sh.at[pl.ds(off, _WB)]))
    plsc.subcore_barrier()

    bufs = ((si_a, di_a, us_a, ud_a, sem_ia, sem_ja, sem_ga, sem_ha),
            (si_b, di_b, us_b, ud_b, sem_ib, sem_jb, sem_gb, sem_hb))

    def istart(j, p):
        si, di, _, _, s_i, s_j, _, _ = bufs[p]
        b = wbase + j * _CHD
        pltpu.async_copy(src_hbm.at[pl.ds(b, _CHD)], si, s_i)
        pltpu.async_copy(dst_hbm.at[pl.ds(b, _CHD)], di, s_j)

    def iwait(j, p):
        si, di, _, _, s_i, s_j, _, _ = bufs[p]
        b = wbase + j * _CHD
        pltpu.make_async_copy(src_hbm.at[pl.ds(b, _CHD)], si, s_i).wait()
        pltpu.make_async_copy(dst_hbm.at[pl.ds(b, _CHD)], di, s_j).wait()

    def gstart(p):
        si, di, us, ud, _, _, s_g, s_h = bufs[p]
        pltpu.async_copy(u_hbm.at[si], us, s_g)
        pltpu.async_copy(u_hbm.at[di], ud, s_h)

    def gwait(p):
        si, di, us, ud, _, _, s_g, s_h = bufs[p]
        pltpu.make_async_copy(u_hbm.at[si], us, s_g).wait()
        pltpu.make_async_copy(u_hbm.at[di], ud, s_h).wait()

    def comp_scat(p):
        _, di, us, ud, _, _, _, _ = bufs[p]
        _sq_rows(us, ud, _CHD)
        pltpu.sync_copy(us, acc_sh.at[di], add=True)

    def half(j, x, y, next_idx):
        iwait(j + 1, y)
        gstart(y)
        gwait(x)
        comp_scat(x)
        if next_idx:
            istart(j + 2, x)

    istart(0, 0)
    iwait(0, 0)
    gstart(0)
    istart(1, 1)

    def body(k, cy):
        half(2 * k, 0, 1, True)
        half(2 * k + 1, 1, 0, True)
        return cy

    lax.fori_loop(0, (_NFD - 3) // 2, body, 0)
    half(_NFD - 3, 0, 1, True)
    half(_NFD - 2, 1, 0, False)
    gwait(0)
    tb = wbase + _NFD * _CHD
    pltpu.sync_copy(src_hbm.at[pl.ds(tb, _TLD)], si8)
    pltpu.sync_copy(dst_hbm.at[pl.ds(tb, _TLD)], di8)
    t1 = pltpu.async_copy(u_hbm.at[si8], us8, sem_t)
    t2 = pltpu.async_copy(u_hbm.at[di8], ud8, sem_t)
    comp_scat(0)
    t1.wait()
    t2.wait()
    _sq_rows(us8, ud8, _TLD)
    pltpu.sync_copy(us8, acc_sh.at[di8], add=True)
    plsc.subcore_barrier()

    def wb(off):
        pltpu.sync_copy(acc_sh.at[pl.ds(off, _WB)], zb_v)
        pltpu.sync_copy(zb_v, out_hbm.at[c, pl.ds(off, _WB)])

    _for_sub_blocks(s, wb)


@functools.cache
def _sc_diff():
    return pl.kernel(
        _sc_diff_body,
        out_type=jax.ShapeDtypeStruct((_NC, _N, _D), jnp.float32),
        mesh=_sc_mesh(),
        scratch_types=[
            pltpu.VMEM_SHARED((_N, _D), jnp.float32),
            pltpu.VMEM((_CHD,), jnp.int32),
            pltpu.VMEM((_CHD,), jnp.int32),
            pltpu.VMEM((_CHD,), jnp.int32),
            pltpu.VMEM((_CHD,), jnp.int32),
            pltpu.VMEM((_CHD, _D), jnp.float32),
            pltpu.VMEM((_CHD, _D), jnp.float32),
            pltpu.VMEM((_CHD, _D), jnp.float32),
            pltpu.VMEM((_CHD, _D), jnp.float32),
            pltpu.VMEM((_TLD,), jnp.int32),
            pltpu.VMEM((_TLD,), jnp.int32),
            pltpu.VMEM((_TLD, _D), jnp.float32),
            pltpu.VMEM((_TLD, _D), jnp.float32),
            pltpu.VMEM((_WB, _D), jnp.float32),
        ] + [pltpu.SemaphoreType.DMA] * 9,
    )


_BE = 4000


def _em_kernel(hs, hd, he, eW1, eb1, eW2, eb2, eg, eb_, nW1, nb1, nW2, nb2,
               ng, nb_, he_out, m_out):
    a = hs[...]
    b = hd[...]
    cc = he[...]
    inv = 1.0 / (2 * _D + _DE)
    mu = (jnp.sum(a, 1, keepdims=True) + jnp.sum(b, 1, keepdims=True)
          + jnp.sum(cc, 1, keepdims=True)) * inv
    am = a - mu
    bm = b - mu
    cm = cc - mu
    var = (jnp.sum(am * am, 1, keepdims=True) + jnp.sum(bm * bm, 1, keepdims=True)
           + jnp.sum(cm * cm, 1, keepdims=True)) * inv
    r = lax.rsqrt(var + 1e-5)
    w1 = eW1[...]
    an = am * r * eg[:, :_D] + eb_[:, :_D]
    bn = bm * r * eg[:, _D:2 * _D] + eb_[:, _D:2 * _D]
    cn = cm * r * eg[:, 2 * _D:] + eb_[:, 2 * _D:]
    t = (jnp.dot(an, w1[:_D], preferred_element_type=jnp.float32)
         + jnp.dot(bn, w1[_D:2 * _D], preferred_element_type=jnp.float32)
         + jnp.dot(cn, w1[2 * _D:], preferred_element_type=jnp.float32)
         + eb1[...])
    t = jnp.maximum(t, 0.0)
    hen = cc + jnp.dot(t, eW2[...], preferred_element_type=jnp.float32) + eb2[...]
    he_out[...] = hen

    inv2 = 1.0 / (_D + _DE)
    mu2 = (jnp.sum(a, 1, keepdims=True) + jnp.sum(hen, 1, keepdims=True)) * inv2
    am2 = a - mu2
    cm2 = hen - mu2
    var2 = (jnp.sum(am2 * am2, 1, keepdims=True)
            + jnp.sum(cm2 * cm2, 1, keepdims=True)) * inv2
    r2 = lax.rsqrt(var2 + 1e-5)
    an2 = am2 * r2 * ng[:, :_D] + nb_[:, :_D]
    cn2 = cm2 * r2 * ng[:, _D:] + nb_[:, _D:]
    w1n = nW1[...]
    t2 = (jnp.dot(an2, w1n[:_D], preferred_element_type=jnp.float32)
          + jnp.dot(cn2, w1n[_D:], preferred_element_type=jnp.float32)
          + nb1[...])
    t2 = jnp.maximum(t2, 0.0)
    m_out[...] = jnp.dot(t2, nW2[...], preferred_element_type=jnp.float32) + nb2[...]


def _full(shape):
    return pl.BlockSpec(shape, lambda j: tuple(0 for _ in shape))


def _em_call(hs, hd, he, eW1, eb1, eW2, eb2, eg, eb_, nW1, nb1, nW2, nb2, ng, nb_):
    grid = _E // _BE
    espec = lambda w: pl.BlockSpec((_BE, w), lambda j: (j, 0))
    return pl.pallas_call(
        _em_kernel,
        grid=(grid,),
        in_specs=[
            espec(_D), espec(_D), espec(_DE),
            _full(eW1.shape), _full(eb1.shape), _full(eW2.shape), _full(eb2.shape),
            _full(eg.shape), _full(eb_.shape),
            _full(nW1.shape), _full(nb1.shape), _full(nW2.shape), _full(nb2.shape),
            _full(ng.shape), _full(nb_.shape),
        ],
        out_specs=[espec(_DE), espec(_D)],
        out_shape=[jax.ShapeDtypeStruct((_E, _DE), jnp.float32),
                   jax.ShapeDtypeStruct((_E, _D), jnp.float32)],
    )(hs, hd, he, eW1, eb1, eW2, eb2, eg, eb_, nW1, nb1, nW2, nb2, ng, nb_)


_BN = 2000


def _u_kernel(h, aggP, degP, Wu1, bu1, Wu2, bu2, u_out, deg_out):
    agg = aggP[0] + aggP[1]
    deg = jnp.maximum(degP[0] + degP[1], 1.0)
    aggn = agg / deg
    hh = h[...]
    w1 = Wu1[...]
    t = (jnp.dot(hh, w1[:_D], preferred_element_type=jnp.float32)
         + jnp.dot(aggn, w1[_D:], preferred_element_type=jnp.float32)
         + bu1[...])
    t = jnp.maximum(t, 0.0)
    u_out[...] = jnp.dot(t, Wu2[...], preferred_element_type=jnp.float32) + bu2[...]
    deg_out[...] = deg


def _u_call(h, aggP, degP, Wu1, bu1, Wu2, bu2):
    grid = _N // _BN
    nspec = pl.BlockSpec((_BN, _D), lambda j: (j, 0))
    pspec = pl.BlockSpec((_NC, _BN, _D), lambda j: (0, j, 0))
    return pl.pallas_call(
        _u_kernel,
        grid=(grid,),
        in_specs=[
            nspec, pspec, pspec,
            _full(Wu1.shape), _full(bu1.shape), _full(Wu2.shape), _full(bu2.shape),
        ],
        out_specs=[nspec, nspec],
        out_shape=[jax.ShapeDtypeStruct((_N, _D), jnp.float32),
                   jax.ShapeDtypeStruct((_N, _D), jnp.float32)],
    )(h, aggP, degP, Wu1, bu1, Wu2, bu2)


def _h_kernel(h, u, diffP, degb, out):
    dsum = diffP[0] + diffP[1]
    tau = jnp.tanh(dsum / degb[...])
    out[...] = (1.0 - tau) * h[...] + tau * u[...]


def _h_call(h, u, diffP, degb):
    grid = _N // _BN
    nspec = pl.BlockSpec((_BN, _D), lambda j: (j, 0))
    return pl.pallas_call(
        _h_kernel,
        grid=(grid,),
        in_specs=[nspec, nspec,
                  pl.BlockSpec((_NC, _BN, _D), lambda j: (0, j, 0)), nspec],
        out_specs=nspec,
        out_shape=jax.ShapeDtypeStruct((_N, _D), jnp.float32),
    )(h, u, diffP, degb)


def _f_kernel(h, pg, pb, lW, lb, mg, mb, loc_out, glob_out, acc):
    j = pl.program_id(0)
    hh = h[...]
    mu = jnp.mean(hh, 1, keepdims=True)
    hm = hh - mu
    var = jnp.mean(hm * hm, 1, keepdims=True)
    loc = hm * lax.rsqrt(var + 1e-5) * pg[...] + pb[...]
    loc_out[...] = loc

    @pl.when(j == 0)
    def _():
        acc[...] = jnp.zeros_like(acc)

    acc[...] += jnp.sum(loc, 0, keepdims=True)
    pooled = acc[...] * (1.0 / _N)
    g = jnp.dot(pooled, lW[...], preferred_element_type=jnp.float32) + lb[...]
    mu2 = jnp.mean(g, 1, keepdims=True)
    gm = g - mu2
    var2 = jnp.mean(gm * gm, 1, keepdims=True)
    glob_out[...] = gm * lax.rsqrt(var2 + 1e-5) * mg[...] + mb[...]


def _f_call(h, pg, pb, lW, lb, mg, mb):
    grid = _N // _BN
    return pl.pallas_call(
        _f_kernel,
        grid=(grid,),
        in_specs=[pl.BlockSpec((_BN, _D), lambda j: (j, 0)),
                  _full(pg.shape), _full(pb.shape), _full(lW.shape),
                  _full(lb.shape), _full(mg.shape), _full(mb.shape)],
        out_specs=[pl.BlockSpec((_BN, _D), lambda j: (j, 0)),
                   pl.BlockSpec((1, _D), lambda j: (0, 0))],
        out_shape=[jax.ShapeDtypeStruct((_N, _D), jnp.float32),
                   jax.ShapeDtypeStruct((1, _D), jnp.float32)],
        scratch_shapes=[pltpu.VMEM((1, _D), jnp.float32)],
    )(h, pg, pb, lW, lb, mg, mb)


def kernel(h, he, edge_index, eW1, eb1, eW2, eb2, eln_g, eln_b, nWm1, nbm1,
           nWm2, nbm2, nln_g, nln_b, nWu1, nbu1, nWu2, nbu2, pn_g, pn_b,
           lin_W, lin_b, mln_g, mln_b):
    src = edge_index[0]
    dst = edge_index[1]
    row = lambda v: v.reshape(1, -1)
    degP = None
    for i in (0, 1):
        if i == 0:
            hs, hd, degP = _sc_gather(True)(h, src, dst)
        else:
            hs, hd = _sc_gather(False)(h, src, dst)
        he, m = _em_call(hs, hd, he,
                         eW1[i], row(eb1[i]), eW2[i], row(eb2[i]),
                         row(eln_g[i]), row(eln_b[i]),
                         nWm1[i], row(nbm1[i]), nWm2[i], row(nbm2[i]),
                         row(nln_g[i]), row(nln_b[i]))
        aggP = _sc_segsum()(m, dst)
        u, degb = _u_call(h, aggP, degP, nWu1[i], row(nbu1[i]), nWu2[i], row(nbu2[i]))
        diffP = _sc_diff()(u, src, dst)
        h = _h_call(h, u, diffP, degb)
    return _f_call(h, row(pn_g), row(pn_b), lin_W, row(lin_b),
                   row(mln_g), row(mln_b))


# speedup vs baseline: 4.8052x; 1.1179x over previous
import functools

import jax
import jax.numpy as jnp
from jax import lax
from jax.experimental import pallas as pl
from jax.experimental.pallas import tpu as pltpu
from jax.experimental.pallas import tpu_sc as plsc

_N = 10000
_E = 160000
_D = 128
_DE = 16

_NC = 2
_NS = 16
_NW = _NC * _NS

_CH = 128
_EPW = _E // _NW
_NFULL = _EPW // _CH
_TAIL = _EPW - _NFULL * _CH
_CHG = 96
_NFG = _EPW // _CHG
_CHD = 56
_NFD = _EPW // _CHD
_TLD = _EPW - _NFD * _CHD


@functools.cache
def _sc_mesh():
    return plsc.VectorSubcoreMesh(
        core_axis_name="c", subcore_axis_name="s",
        num_cores=_NC, num_subcores=_NS)


_WB = 40
_NBLK = _N // _WB
_BLK_ROUNDS = (_NBLK + _NS - 1) // _NS


def _for_sub_blocks(s, fn):
    for kk in range(_BLK_ROUNDS):
        blk = s + _NS * kk

        @pl.when(blk < _NBLK)
        def _():
            fn(blk * _WB)


def _worker_id():
    c = lax.axis_index("c")
    s = lax.axis_index("s")
    return s * _NC + c, c, s


def _zero_vmem(ref, rows, cols):
    z = jnp.zeros((16,), jnp.float32)

    def body(r, cy):
        for j in range(cols // 16):
            ref[r, pl.ds(j * 16, 16)] = z
        return cy

    lax.fori_loop(0, rows, body, 0)


def _gather_pipeline(h_hbm, src_hbm, dst_hbm, hs_hbm, hd_hbm,
                     is_v, id_v, rs_v, rd_v, is8, id8, r8_v,
                     sem_is, sem_id, sem_gs, sem_gd, sem_ss, sem_sd, sem_t,
                     ch, nfull, on_dst_idx=None, on_dst_idx_tail=None):
    wid, _, _ = _worker_id()
    wb = wid * _EPW

    tb = wb + nfull * ch
    pltpu.sync_copy(src_hbm.at[pl.ds(tb, _TAIL)], is8)
    pltpu.sync_copy(dst_hbm.at[pl.ds(tb, _TAIL)], id8)
    pltpu.async_copy(h_hbm.at[is8], r8_v, sem_t).wait()
    pltpu.sync_copy(r8_v, hs_hbm.at[pl.ds(tb, _TAIL)])
    pltpu.async_copy(h_hbm.at[id8], r8_v, sem_t).wait()
    pltpu.sync_copy(r8_v, hd_hbm.at[pl.ds(tb, _TAIL)])
    if on_dst_idx_tail is not None:
        on_dst_idx_tail(id8)

    def istart(j):
        b = wb + j * ch
        pltpu.async_copy(src_hbm.at[pl.ds(b, ch)], is_v, sem_is)
        pltpu.async_copy(dst_hbm.at[pl.ds(b, ch)], id_v, sem_id)

    def iwait(j):
        b = wb + j * ch
        pltpu.make_async_copy(src_hbm.at[pl.ds(b, ch)], is_v, sem_is).wait()
        pltpu.make_async_copy(dst_hbm.at[pl.ds(b, ch)], id_v, sem_id).wait()

    def swait(j):
        b = wb + j * ch
        pltpu.make_async_copy(rs_v, hs_hbm.at[pl.ds(b, ch)], sem_ss).wait()
        pltpu.make_async_copy(rd_v, hd_hbm.at[pl.ds(b, ch)], sem_sd).wait()

    istart(0)

    def body(j, cy):
        b = wb + j * ch
        iwait(j)
        if on_dst_idx is not None:
            on_dst_idx(id_v)

        @pl.when(j > 0)
        def _():
            swait(j - 1)

        g1 = pltpu.async_copy(h_hbm.at[is_v], rs_v, sem_gs)
        g2 = pltpu.async_copy(h_hbm.at[id_v], rd_v, sem_gd)
        g1.wait()
        g2.wait()

        @pl.when(j < nfull - 1)
        def _():
            istart(j + 1)

        pltpu.async_copy(rs_v, hs_hbm.at[pl.ds(b, ch)], sem_ss)
        pltpu.async_copy(rd_v, hd_hbm.at[pl.ds(b, ch)], sem_sd)
        return cy

    lax.fori_loop(0, nfull, body, 0)
    swait(nfull - 1)


def _sc_gather_deg_body(h_hbm, src_hbm, dst_hbm, hs_hbm, hd_hbm, deg_hbm,
                        acc_sh, is_v, id_v, rs_v, rd_v, is8, id8, r8_v, ones_v,
                        zb_v, sem_is, sem_id, sem_gs, sem_gd, sem_ss, sem_sd,
                        sem_t):
    wid, c, s = _worker_id()
    _zero_vmem(zb_v, _WB, _D)
    _for_sub_blocks(s, lambda off: pltpu.sync_copy(zb_v, acc_sh.at[pl.ds(off, _WB)]))

    def fill_ones(r, cy):
        one = jnp.ones((16,), jnp.float32)
        for j in range(_D // 16):
            ones_v[r, pl.ds(j * 16, 16)] = one
        return cy

    lax.fori_loop(0, _CHG, fill_ones, 0)
    plsc.subcore_barrier()

    _gather_pipeline(
        h_hbm, src_hbm, dst_hbm, hs_hbm, hd_hbm,
        is_v, id_v, rs_v, rd_v, is8, id8, r8_v,
        sem_is, sem_id, sem_gs, sem_gd, sem_ss, sem_sd, sem_t,
        _CHG, _NFG,
        on_dst_idx=lambda idx: pltpu.sync_copy(ones_v, acc_sh.at[idx], add=True),
        on_dst_idx_tail=lambda idx: pltpu.sync_copy(
            ones_v.at[pl.ds(0, _TAIL)], acc_sh.at[idx], add=True),
    )
    plsc.subcore_barrier()

    def wbk(off):
        pltpu.sync_copy(acc_sh.at[pl.ds(off, _WB)], zb_v)
        pltpu.sync_copy(zb_v, deg_hbm.at[c, pl.ds(off, _WB)])

    _for_sub_blocks(s, wbk)


def _sc_gather_body(h_hbm, src_hbm, dst_hbm, hs_hbm, hd_hbm,
                    is_v, id_v, rs_v, rd_v, is8, id8, r8_v,
                    sem_is, sem_id, sem_gs, sem_gd, sem_ss, sem_sd, sem_t):
    _gather_pipeline(h_hbm, src_hbm, dst_hbm, hs_hbm, hd_hbm,
                     is_v, id_v, rs_v, rd_v, is8, id8, r8_v,
                     sem_is, sem_id, sem_gs, sem_gd, sem_ss, sem_sd, sem_t,
                     _CH, _NFULL)


_GATHER_SCRATCH = [
    pltpu.VMEM((_CH,), jnp.int32),
    pltpu.VMEM((_CH,), jnp.int32),
    pltpu.VMEM((_CH, _D), jnp.float32),
    pltpu.VMEM((_CH, _D), jnp.float32),
    pltpu.VMEM((_TAIL,), jnp.int32),
    pltpu.VMEM((_TAIL,), jnp.int32),
    pltpu.VMEM((_TAIL, _D), jnp.float32),
] + [pltpu.SemaphoreType.DMA] * 7


@functools.cache
def _sc_gather(with_deg):
    if with_deg:
        return pl.kernel(
            _sc_gather_deg_body,
            out_type=(jax.ShapeDtypeStruct((_E, _D), jnp.float32),
                      jax.ShapeDtypeStruct((_E, _D), jnp.float32),
                      jax.ShapeDtypeStruct((_NC, _N, _D), jnp.float32)),
            mesh=_sc_mesh(),
            scratch_types=(
                [pltpu.VMEM_SHARED((_N, _D), jnp.float32),
                 pltpu.VMEM((_CHG,), jnp.int32),
                 pltpu.VMEM((_CHG,), jnp.int32),
                 pltpu.VMEM((_CHG, _D), jnp.float32),
                 pltpu.VMEM((_CHG, _D), jnp.float32),
                 pltpu.VMEM((_TAIL,), jnp.int32),
                 pltpu.VMEM((_TAIL,), jnp.int32),
                 pltpu.VMEM((_TAIL, _D), jnp.float32),
                 pltpu.VMEM((_CHG, _D), jnp.float32),
                 pltpu.VMEM((_WB, _D), jnp.float32)]
                + [pltpu.SemaphoreType.DMA] * 7
            ),
        )
    return pl.kernel(
        _sc_gather_body,
        out_type=(jax.ShapeDtypeStruct((_E, _D), jnp.float32),
                  jax.ShapeDtypeStruct((_E, _D), jnp.float32)),
        mesh=_sc_mesh(),
        scratch_types=list(_GATHER_SCRATCH),
    )


def _sc_segsum_body(vals_hbm, dst_hbm, out_hbm, acc_sh,
                    rows_a, rows_b, idx_a, idx_b, rows8, idx8, zb_v,
                    sem_ra, sem_rb, sem_ia, sem_ib):
    wid, c, s = _worker_id()
    wbase = wid * _EPW
    _zero_vmem(zb_v, _WB, _D)
    _for_sub_blocks(s, lambda off: pltpu.sync_copy(zb_v, acc_sh.at[pl.ds(off, _WB)]))
    plsc.subcore_barrier()

    bufs = ((rows_a, idx_a, sem_ra, sem_ia), (rows_b, idx_b, sem_rb, sem_ib))

    def lstart(j, p):
        rv, iv, sr, si = bufs[p]
        b = wbase + j * _CH
        pltpu.async_copy(vals_hbm.at[pl.ds(b, _CH)], rv, sr)
        pltpu.async_copy(dst_hbm.at[pl.ds(b, _CH)], iv, si)

    def lwait(j, p):
        rv, iv, sr, si = bufs[p]
        b = wbase + j * _CH
        pltpu.make_async_copy(vals_hbm.at[pl.ds(b, _CH)], rv, sr).wait()
        pltpu.make_async_copy(dst_hbm.at[pl.ds(b, _CH)], iv, si).wait()

    def scatter(p):
        rv, iv, _, _ = bufs[p]
        pltpu.sync_copy(rv, acc_sh.at[iv], add=True)

    lstart(0, 0)

    def body(k, cy):
        lwait(2 * k, 0)
        lstart(2 * k + 1, 1)
        scatter(0)
        lwait(2 * k + 1, 1)
        lstart(2 * k + 2, 0)
        scatter(1)
        return cy

    lax.fori_loop(0, (_NFULL - 1) // 2, body, 0)
    lwait(_NFULL - 1, 0)
    tb = wbase + _NFULL * _CH
    pltpu.sync_copy(vals_hbm.at[pl.ds(tb, _TAIL)], rows8)
    pltpu.sync_copy(dst_hbm.at[pl.ds(tb, _TAIL)], idx8)
    scatter(0)
    pltpu.sync_copy(rows8, acc_sh.at[idx8], add=True)
    plsc.subcore_barrier()

    def wb(off):
        pltpu.sync_copy(acc_sh.at[pl.ds(off, _WB)], zb_v)
        pltpu.sync_copy(zb_v, out_hbm.at[c, pl.ds(off, _WB)])

    _for_sub_blocks(s, wb)


@functools.cache
def _sc_segsum():
    return pl.kernel(
        _sc_segsum_body,
        out_type=jax.ShapeDtypeStruct((_NC, _N, _D), jnp.float32),
        mesh=_sc_mesh(),
        scratch_types=[
            pltpu.VMEM_SHARED((_N, _D), jnp.float32),
            pltpu.VMEM((_CH, _D), jnp.float32),
            pltpu.VMEM((_CH, _D), jnp.float32),
            pltpu.VMEM((_CH,), jnp.int32),
            pltpu.VMEM((_CH,), jnp.int32),
            pltpu.VMEM((_TAIL, _D), jnp.float32),
            pltpu.VMEM((_TAIL,), jnp.int32),
            pltpu.VMEM((_WB, _D), jnp.float32),
        ] + [pltpu.SemaphoreType.DMA] * 4,
    )


def _sq_rows(us, ud, nrows):
    def row(r, rcy):
        for jj in range(_D // 16):
            sl = pl.ds(jj * 16, 16)
            d = us[r, sl] - ud[r, sl]
            us[r, sl] = d * d
        return rcy

    lax.fori_loop(0, nrows, row, 0)


def _sc_diff_body(u_hbm, src_hbm, dst_hbm, out_hbm, acc_sh,
                  si_a, di_a, si_b, di_b, us_a, ud_a, us_b, ud_b,
                  si8, di8, us8, ud8, zb_v,
                  sem_ia, sem_ib, sem_ja, sem_jb,
                  sem_ga, sem_gb, sem_ha, sem_hb, sem_t):
    wid, c, s = _worker_id()
    wbase = wid * _EPW
    _zero_vmem(zb_v, _WB, _D)
    _for_sub_blocks(s, lambda off: pltpu.sync_copy(zb_v, acc_sh.at[pl.ds(off, _WB)]))
    plsc.subcore_barrier()

    bufs = ((si_a, di_a, us_a, ud_a, sem_ia, sem_ja, sem_ga, sem_ha),
            (si_b, di_b, us_b, ud_b, sem_ib, sem_jb, sem_gb, sem_hb))

    def istart(j, p):
        si, di, _, _, s_i, s_j, _, _ = bufs[p]
        b = wbase + j * _CHD
        pltpu.async_copy(src_hbm.at[pl.ds(b, _CHD)], si, s_i)
        pltpu.async_copy(dst_hbm.at[pl.ds(b, _CHD)], di, s_j)

    def iwait(j, p):
        si, di, _, _, s_i, s_j, _, _ = bufs[p]
        b = wbase + j * _CHD
        pltpu.make_async_copy(src_hbm.at[pl.ds(b, _CHD)], si, s_i).wait()
        pltpu.make_async_copy(dst_hbm.at[pl.ds(b, _CHD)], di, s_j).wait()

    def gstart(p):
        si, di, us, ud, _, _, s_g, s_h = bufs[p]
        pltpu.async_copy(u_hbm.at[si], us, s_g)
        pltpu.async_copy(u_hbm.at[di], ud, s_h)

    def gwait(p):
        si, di, us, ud, _, _, s_g, s_h = bufs[p]
        pltpu.make_async_copy(u_hbm.at[si], us, s_g).wait()
        pltpu.make_async_copy(u_hbm.at[di], ud, s_h).wait()

    def comp_scat(p):
        _, di, us, ud, _, _, _, _ = bufs[p]
        _sq_rows(us, ud, _CHD)
        pltpu.sync_copy(us, acc_sh.at[di], add=True)

    def half(j, x, y, next_idx):
        iwait(j + 1, y)
        gstart(y)
        gwait(x)
        comp_scat(x)
        if next_idx:
            istart(j + 2, x)

    istart(0, 0)
    iwait(0, 0)
    gstart(0)
    istart(1, 1)

    def body(k, cy):
        half(2 * k, 0, 1, True)
        half(2 * k + 1, 1, 0, True)
        return cy

    lax.fori_loop(0, (_NFD - 3) // 2, body, 0)
    half(_NFD - 3, 0, 1, True)
    half(_NFD - 2, 1, 0, False)
    gwait(0)
    tb = wbase + _NFD * _CHD
    pltpu.sync_copy(src_hbm.at[pl.ds(tb, _TLD)], si8)
    pltpu.sync_copy(dst_hbm.at[pl.ds(tb, _TLD)], di8)
    t1 = pltpu.async_copy(u_hbm.at[si8], us8, sem_t)
    t2 = pltpu.async_copy(u_hbm.at[di8], ud8, sem_t)
    comp_scat(0)
    t1.wait()
    t2.wait()
    _sq_rows(us8, ud8, _TLD)
    pltpu.sync_copy(us8, acc_sh.at[di8], add=True)
    plsc.subcore_barrier()

    def wb(off):
        pltpu.sync_copy(acc_sh.at[pl.ds(off, _WB)], zb_v)
        pltpu.sync_copy(zb_v, out_hbm.at[c, pl.ds(off, _WB)])

    _for_sub_blocks(s, wb)


@functools.cache
def _sc_diff():
    return pl.kernel(
        _sc_diff_body,
        out_type=jax.ShapeDtypeStruct((_NC, _N, _D), jnp.float32),
        mesh=_sc_mesh(),
        scratch_types=[
            pltpu.VMEM_SHARED((_N, _D), jnp.float32),
            pltpu.VMEM((_CHD,), jnp.int32),
            pltpu.VMEM((_CHD,), jnp.int32),
            pltpu.VMEM((_CHD,), jnp.int32),
            pltpu.VMEM((_CHD,), jnp.int32),
            pltpu.VMEM((_CHD, _D), jnp.float32),
            pltpu.VMEM((_CHD, _D), jnp.float32),
            pltpu.VMEM((_CHD, _D), jnp.float32),
            pltpu.VMEM((_CHD, _D), jnp.float32),
            pltpu.VMEM((_TLD,), jnp.int32),
            pltpu.VMEM((_TLD,), jnp.int32),
            pltpu.VMEM((_TLD, _D), jnp.float32),
            pltpu.VMEM((_TLD, _D), jnp.float32),
            pltpu.VMEM((_WB, _D), jnp.float32),
        ] + [pltpu.SemaphoreType.DMA] * 9,
    )


_BE = 4000


def _em_kernel(hs, hd, he, W1g, s1, b1p, eW2, eb2, Wm1g, s2, b2p, nW2, nb2,
               he_out, m_out):
    a = hs[...]
    b = hd[...]
    cc = he[...]
    sa = jnp.sum(a, 1, keepdims=True)
    qa = jnp.sum(a * a, 1, keepdims=True)
    inv = 1.0 / (2 * _D + _DE)
    mu = (sa + jnp.sum(b, 1, keepdims=True) + jnp.sum(cc, 1, keepdims=True)) * inv
    q = (qa + jnp.sum(b * b, 1, keepdims=True)
         + jnp.sum(cc * cc, 1, keepdims=True)) * inv
    r = lax.rsqrt(q - mu * mu + 1e-5)
    w = W1g[...]
    acc = (jnp.dot(a, w[:_D], preferred_element_type=jnp.float32)
           + jnp.dot(b, w[_D:2 * _D], preferred_element_type=jnp.float32)
           + jnp.dot(cc, w[2 * _D:], preferred_element_type=jnp.float32))
    t = jnp.maximum(r * acc - (r * mu) * s1[...] + b1p[...], 0.0)
    hen = cc + jnp.dot(t, eW2[...], preferred_element_type=jnp.float32) + eb2[...]
    he_out[...] = hen

    inv2 = 1.0 / (_D + _DE)
    mu2 = (sa + jnp.sum(hen, 1, keepdims=True)) * inv2
    q2 = (qa + jnp.sum(hen * hen, 1, keepdims=True)) * inv2
    r2 = lax.rsqrt(q2 - mu2 * mu2 + 1e-5)
    w2 = Wm1g[...]
    acc2 = (jnp.dot(a, w2[:_D], preferred_element_type=jnp.float32)
            + jnp.dot(hen, w2[_D:], preferred_element_type=jnp.float32))
    t2 = jnp.maximum(r2 * acc2 - (r2 * mu2) * s2[...] + b2p[...], 0.0)
    m_out[...] = jnp.dot(t2, nW2[...], preferred_element_type=jnp.float32) + nb2[...]


def _full(shape):
    return pl.BlockSpec(shape, lambda j: tuple(0 for _ in shape))


def _em_call(hs, hd, he, W1g, s1, b1p, eW2, eb2, Wm1g, s2, b2p, nW2, nb2):
    grid = _E // _BE
    espec = lambda w: pl.BlockSpec((_BE, w), lambda j: (j, 0))
    return pl.pallas_call(
        _em_kernel,
        grid=(grid,),
        in_specs=[
            espec(_D), espec(_D), espec(_DE),
            _full(W1g.shape), _full(s1.shape), _full(b1p.shape),
            _full(eW2.shape), _full(eb2.shape),
            _full(Wm1g.shape), _full(s2.shape), _full(b2p.shape),
            _full(nW2.shape), _full(nb2.shape),
        ],
        out_specs=[espec(_DE), espec(_D)],
        out_shape=[jax.ShapeDtypeStruct((_E, _DE), jnp.float32),
                   jax.ShapeDtypeStruct((_E, _D), jnp.float32)],
    )(hs, hd, he, W1g, s1, b1p, eW2, eb2, Wm1g, s2, b2p, nW2, nb2)


_BN = 2000


def _u_kernel(h, aggP, degP, Wu1, bu1, Wu2, bu2, u_out, deg_out):
    agg = aggP[0] + aggP[1]
    deg = jnp.maximum(degP[0] + degP[1], 1.0)
    aggn = agg / deg
    hh = h[...]
    w1 = Wu1[...]
    t = (jnp.dot(hh, w1[:_D], preferred_element_type=jnp.float32)
         + jnp.dot(aggn, w1[_D:], preferred_element_type=jnp.float32)
         + bu1[...])
    t = jnp.maximum(t, 0.0)
    u_out[...] = jnp.dot(t, Wu2[...], preferred_element_type=jnp.float32) + bu2[...]
    deg_out[...] = deg


def _u_call(h, aggP, degP, Wu1, bu1, Wu2, bu2):
    grid = _N // _BN
    nspec = pl.BlockSpec((_BN, _D), lambda j: (j, 0))
    pspec = pl.BlockSpec((_NC, _BN, _D), lambda j: (0, j, 0))
    return pl.pallas_call(
        _u_kernel,
        grid=(grid,),
        in_specs=[
            nspec, pspec, pspec,
            _full(Wu1.shape), _full(bu1.shape), _full(Wu2.shape), _full(bu2.shape),
        ],
        out_specs=[nspec, nspec],
        out_shape=[jax.ShapeDtypeStruct((_N, _D), jnp.float32),
                   jax.ShapeDtypeStruct((_N, _D), jnp.float32)],
    )(h, aggP, degP, Wu1, bu1, Wu2, bu2)


def _h_kernel(h, u, diffP, degb, out):
    dsum = diffP[0] + diffP[1]
    tau = jnp.tanh(dsum / degb[...])
    out[...] = (1.0 - tau) * h[...] + tau * u[...]


def _h_call(h, u, diffP, degb):
    grid = _N // _BN
    nspec = pl.BlockSpec((_BN, _D), lambda j: (j, 0))
    return pl.pallas_call(
        _h_kernel,
        grid=(grid,),
        in_specs=[nspec, nspec,
                  pl.BlockSpec((_NC, _BN, _D), lambda j: (0, j, 0)), nspec],
        out_specs=nspec,
        out_shape=jax.ShapeDtypeStruct((_N, _D), jnp.float32),
    )(h, u, diffP, degb)


def _f_kernel(h, pg, pb, lW, lb, mg, mb, loc_out, glob_out, acc):
    j = pl.program_id(0)
    hh = h[...]
    mu = jnp.mean(hh, 1, keepdims=True)
    hm = hh - mu
    var = jnp.mean(hm * hm, 1, keepdims=True)
    loc = hm * lax.rsqrt(var + 1e-5) * pg[...] + pb[...]
    loc_out[...] = loc

    @pl.when(j == 0)
    def _():
        acc[...] = jnp.zeros_like(acc)

    acc[...] += jnp.sum(loc, 0, keepdims=True)
    pooled = acc[...] * (1.0 / _N)
    g = jnp.dot(pooled, lW[...], preferred_element_type=jnp.float32) + lb[...]
    mu2 = jnp.mean(g, 1, keepdims=True)
    gm = g - mu2
    var2 = jnp.mean(gm * gm, 1, keepdims=True)
    glob_out[...] = gm * lax.rsqrt(var2 + 1e-5) * mg[...] + mb[...]


def _f_call(h, pg, pb, lW, lb, mg, mb):
    grid = _N // _BN
    return pl.pallas_call(
        _f_kernel,
        grid=(grid,),
        in_specs=[pl.BlockSpec((_BN, _D), lambda j: (j, 0)),
                  _full(pg.shape), _full(pb.shape), _full(lW.shape),
                  _full(lb.shape), _full(mg.shape), _full(mb.shape)],
        out_specs=[pl.BlockSpec((_BN, _D), lambda j: (j, 0)),
                   pl.BlockSpec((1, _D), lambda j: (0, 0))],
        out_shape=[jax.ShapeDtypeStruct((_N, _D), jnp.float32),
                   jax.ShapeDtypeStruct((1, _D), jnp.float32)],
        scratch_shapes=[pltpu.VMEM((1, _D), jnp.float32)],
    )(h, pg, pb, lW, lb, mg, mb)


def kernel(h, he, edge_index, eW1, eb1, eW2, eb2, eln_g, eln_b, nWm1, nbm1,
           nWm2, nbm2, nln_g, nln_b, nWu1, nbu1, nWu2, nbu2, pn_g, pn_b,
           lin_W, lin_b, mln_g, mln_b):
    src = edge_index[0]
    dst = edge_index[1]
    row = lambda v: v.reshape(1, -1)
    degP = None
    for i in (0, 1):
        if i == 0:
            hs, hd, degP = _sc_gather(True)(h, src, dst)
        else:
            hs, hd = _sc_gather(False)(h, src, dst)
        W1g = eW1[i] * eln_g[i][:, None]
        s1 = jnp.sum(W1g, 0)
        b1p = eb1[i] + eln_b[i] @ eW1[i]
        Wm1g = nWm1[i] * nln_g[i][:, None]
        s2 = jnp.sum(Wm1g, 0)
        b2p = nbm1[i] + nln_b[i] @ nWm1[i]
        he, m = _em_call(hs, hd, he,
                         W1g, row(s1), row(b1p), eW2[i], row(eb2[i]),
                         Wm1g, row(s2), row(b2p), nWm2[i], row(nbm2[i]))
        aggP = _sc_segsum()(m, dst)
        u, degb = _u_call(h, aggP, degP, nWu1[i], row(nbu1[i]), nWu2[i], row(nbu2[i]))
        diffP = _sc_diff()(u, src, dst)
        h = _h_call(h, u, diffP, degb)
    return _f_call(h, row(pn_g), row(pn_b), lin_W, row(lin_b),
                   row(mln_g), row(mln_b))


# speedup vs baseline: 5.3094x; 1.1049x over previous
import functools

import jax
import jax.numpy as jnp
from jax import lax
from jax.experimental import pallas as pl
from jax.experimental.pallas import tpu as pltpu
from jax.experimental.pallas import tpu_sc as plsc

_N = 10000
_E = 160000
_D = 128
_DE = 16

_NC = 2
_NS = 16
_NW = _NC * _NS

_CH = 128
_EPW = _E // _NW
_EH = _E // 2
_NCHH = _EH // _CH
_NPW = _NCHH // _NW
_XW = _NCHH - _NPW * _NW
_CHG = 80
_NCHG = _EH // _CHG
_NPWG = _NCHG // _NW
_XWG = _NCHG - _NPWG * _NW
_CHD = 56
_NFD = _EPW // _CHD
_TLD = _EPW - _NFD * _CHD


@functools.cache
def _sc_mesh():
    return plsc.VectorSubcoreMesh(
        core_axis_name="c", subcore_axis_name="s",
        num_cores=_NC, num_subcores=_NS)


_WB = 40
_NBLK = _N // _WB
_BLK_ROUNDS = (_NBLK + _NS - 1) // _NS


def _for_sub_blocks(s, fn):
    for kk in range(_BLK_ROUNDS):
        blk = s + _NS * kk

        @pl.when(blk < _NBLK)
        def _():
            fn(blk * _WB)


def _worker_id():
    c = lax.axis_index("c")
    s = lax.axis_index("s")
    return s * _NC + c, c, s


def _zero_vmem(ref, rows, cols):
    z = jnp.zeros((16,), jnp.float32)

    def body(r, cy):
        for j in range(cols // 16):
            ref[r, pl.ds(j * 16, 16)] = z
        return cy

    lax.fori_loop(0, rows, body, 0)


def _gather_pipeline(h_hbm, src_hbm, dst_hbm, hs_hbm, hd_hbm,
                     is_v, id_v, rs_v, rd_v,
                     sem_is, sem_id, sem_gs, sem_gd, sem_ss, sem_sd,
                     off, ch, npw, xw, on_dst_idx=None):
    wid, _, _ = _worker_id()
    nch = npw + jnp.where(wid < xw, 1, 0)

    def cbase(j):
        return (wid + _NW * j) * ch

    def istart(j):
        b = off + cbase(j)
        pltpu.async_copy(src_hbm.at[pl.ds(b, ch)], is_v, sem_is)
        pltpu.async_copy(dst_hbm.at[pl.ds(b, ch)], id_v, sem_id)

    def iwait(j):
        b = off + cbase(j)
        pltpu.make_async_copy(src_hbm.at[pl.ds(b, ch)], is_v, sem_is).wait()
        pltpu.make_async_copy(dst_hbm.at[pl.ds(b, ch)], id_v, sem_id).wait()

    def swait(j):
        b = cbase(j)
        pltpu.make_async_copy(rs_v, hs_hbm.at[pl.ds(b, ch)], sem_ss).wait()
        pltpu.make_async_copy(rd_v, hd_hbm.at[pl.ds(b, ch)], sem_sd).wait()

    istart(0)

    def body(j, cy):
        b = cbase(j)
        iwait(j)
        if on_dst_idx is not None:
            on_dst_idx(id_v)

        @pl.when(j > 0)
        def _():
            swait(j - 1)

        g1 = pltpu.async_copy(h_hbm.at[is_v], rs_v, sem_gs)
        g2 = pltpu.async_copy(h_hbm.at[id_v], rd_v, sem_gd)
        g1.wait()
        g2.wait()

        @pl.when(j < nch - 1)
        def _():
            istart(j + 1)

        pltpu.async_copy(rs_v, hs_hbm.at[pl.ds(b, ch)], sem_ss)
        pltpu.async_copy(rd_v, hd_hbm.at[pl.ds(b, ch)], sem_sd)
        return cy

    lax.fori_loop(0, nch, body, 0)
    swait(nch - 1)


@functools.cache
def _sc_gather(half, with_deg):
    off = half * _EH
    if with_deg:
        def deg_body(h_hbm, src_hbm, dst_hbm, hs_hbm, hd_hbm, deg_hbm,
                     acc_sh, is_v, id_v, rs_v, rd_v, ones_v,
                     zb_v, sem_is, sem_id, sem_gs, sem_gd, sem_ss, sem_sd):
            _, c, s = _worker_id()
            _zero_vmem(zb_v, _WB, _D)
            _for_sub_blocks(
                s, lambda o: pltpu.sync_copy(zb_v, acc_sh.at[pl.ds(o, _WB)]))

            def fill_ones(r, cy):
                one = jnp.ones((16,), jnp.float32)
                for j in range(_D // 16):
                    ones_v[r, pl.ds(j * 16, 16)] = one
                return cy

            lax.fori_loop(0, _CHG, fill_ones, 0)
            plsc.subcore_barrier()

            _gather_pipeline(
                h_hbm, src_hbm, dst_hbm, hs_hbm, hd_hbm,
                is_v, id_v, rs_v, rd_v,
                sem_is, sem_id, sem_gs, sem_gd, sem_ss, sem_sd,
                off, _CHG, _NPWG, _XWG,
                on_dst_idx=lambda i_: pltpu.sync_copy(
                    ones_v, acc_sh.at[i_], add=True),
            )
            plsc.subcore_barrier()

            def wbk(o):
                pltpu.sync_copy(acc_sh.at[pl.ds(o, _WB)], zb_v)
                pltpu.sync_copy(zb_v, deg_hbm.at[c, pl.ds(o, _WB)])

            _for_sub_blocks(s, wbk)

        return pl.kernel(
            deg_body,
            out_type=(jax.ShapeDtypeStruct((_EH, _D), jnp.float32),
                      jax.ShapeDtypeStruct((_EH, _D), jnp.float32),
                      jax.ShapeDtypeStruct((_NC, _N, _D), jnp.float32)),
            mesh=_sc_mesh(),
            scratch_types=(
                [pltpu.VMEM_SHARED((_N, _D), jnp.float32),
                 pltpu.VMEM((_CHG,), jnp.int32),
                 pltpu.VMEM((_CHG,), jnp.int32),
                 pltpu.VMEM((_CHG, _D), jnp.float32),
                 pltpu.VMEM((_CHG, _D), jnp.float32),
                 pltpu.VMEM((_CHG, _D), jnp.float32),
                 pltpu.VMEM((_WB, _D), jnp.float32)]
                + [pltpu.SemaphoreType.DMA] * 6
            ),
        )

    def body(h_hbm, src_hbm, dst_hbm, hs_hbm, hd_hbm,
             is_v, id_v, rs_v, rd_v,
             sem_is, sem_id, sem_gs, sem_gd, sem_ss, sem_sd):
        _gather_pipeline(h_hbm, src_hbm, dst_hbm, hs_hbm, hd_hbm,
                         is_v, id_v, rs_v, rd_v,
                         sem_is, sem_id, sem_gs, sem_gd, sem_ss, sem_sd,
                         off, _CH, _NPW, _XW)

    return pl.kernel(
        body,
        out_type=(jax.ShapeDtypeStruct((_EH, _D), jnp.float32),
                  jax.ShapeDtypeStruct((_EH, _D), jnp.float32)),
        mesh=_sc_mesh(),
        scratch_types=[
            pltpu.VMEM((_CH,), jnp.int32),
            pltpu.VMEM((_CH,), jnp.int32),
            pltpu.VMEM((_CH, _D), jnp.float32),
            pltpu.VMEM((_CH, _D), jnp.float32),
        ] + [pltpu.SemaphoreType.DMA] * 6,
    )


@functools.cache
def _sc_segsum(half):
    off = half * _EH

    def body_fn(vals_hbm, dst_hbm, out_hbm, acc_sh,
                rows_a, rows_b, idx_a, idx_b, zb_v,
                sem_ra, sem_rb, sem_ia, sem_ib):
        wid, c, s = _worker_id()
        _zero_vmem(zb_v, _WB, _D)
        _for_sub_blocks(
            s, lambda o: pltpu.sync_copy(zb_v, acc_sh.at[pl.ds(o, _WB)]))
        plsc.subcore_barrier()

        bufs = ((rows_a, idx_a, sem_ra, sem_ia), (rows_b, idx_b, sem_rb, sem_ib))

        def vb(k):
            return (wid + _NW * k) * _CH

        def lstart(k, p):
            rv, iv, sr, si = bufs[p]
            pltpu.async_copy(vals_hbm.at[pl.ds(vb(k), _CH)], rv, sr)
            pltpu.async_copy(dst_hbm.at[pl.ds(off + vb(k), _CH)], iv, si)

        def lwait(k, p):
            rv, iv, sr, si = bufs[p]
            pltpu.make_async_copy(
                vals_hbm.at[pl.ds(vb(k), _CH)], rv, sr).wait()
            pltpu.make_async_copy(
                dst_hbm.at[pl.ds(off + vb(k), _CH)], iv, si).wait()

        def scatter(p):
            rv, iv, _, _ = bufs[p]
            pltpu.sync_copy(rv, acc_sh.at[iv], add=True)

        lstart(0, 0)

        def body(k, cy):
            lwait(2 * k, 0)
            lstart(2 * k + 1, 1)
            scatter(0)
            lwait(2 * k + 1, 1)
            lstart(2 * k + 2, 0)
            scatter(1)
            return cy

        lax.fori_loop(0, (_NPW - 1) // 2, body, 0)
        lwait(_NPW - 1, 0)

        @pl.when(wid < _XW)
        def _():
            lstart(_NPW, 1)

        scatter(0)

        @pl.when(wid < _XW)
        def _():
            lwait(_NPW, 1)
            scatter(1)

        plsc.subcore_barrier()

        def wb(o):
            pltpu.sync_copy(acc_sh.at[pl.ds(o, _WB)], zb_v)
            pltpu.sync_copy(zb_v, out_hbm.at[c, pl.ds(o, _WB)])

        _for_sub_blocks(s, wb)

    return pl.kernel(
        body_fn,
        out_type=jax.ShapeDtypeStruct((_NC, _N, _D), jnp.float32),
        mesh=_sc_mesh(),
        scratch_types=[
            pltpu.VMEM_SHARED((_N, _D), jnp.float32),
            pltpu.VMEM((_CH, _D), jnp.float32),
            pltpu.VMEM((_CH, _D), jnp.float32),
            pltpu.VMEM((_CH,), jnp.int32),
            pltpu.VMEM((_CH,), jnp.int32),
            pltpu.VMEM((_WB, _D), jnp.float32),
        ] + [pltpu.SemaphoreType.DMA] * 4,
    )


def _sq_rows(us, ud, nrows):
    def row(r, rcy):
        for jj in range(_D // 16):
            sl = pl.ds(jj * 16, 16)
            d = us[r, sl] - ud[r, sl]
            us[r, sl] = d * d
        return rcy

    lax.fori_loop(0, nrows, row, 0)


def _sc_diff_body(u_hbm, src_hbm, dst_hbm, out_hbm, acc_sh,
                  si_a, di_a, si_b, di_b, us_a, ud_a, us_b, ud_b,
                  si8, di8, us8, ud8, zb_v,
                  sem_ia, sem_ib, sem_ja, sem_jb,
                  sem_ga, sem_gb, sem_ha, sem_hb, sem_t):
    wid, c, s = _worker_id()
    wbase = wid * _EPW
    _zero_vmem(zb_v, _WB, _D)
    _for_sub_blocks(s, lambda off: pltpu.sync_copy(zb_v, acc_sh.at[pl.ds(off, _WB)]))
    plsc.subcore_barrier()

    bufs = ((si_a, di_a, us_a, ud_a, sem_ia, sem_ja, sem_ga, sem_ha),
            (si_b, di_b, us_b, ud_b, sem_ib, sem_jb, sem_gb, sem_hb))

    def istart(j, p):
        si, di, _, _, s_i, s_j, _, _ = bufs[p]
        b = wbase + j * _CHD
        pltpu.async_copy(src_hbm.at[pl.ds(b, _CHD)], si, s_i)
        pltpu.async_copy(dst_hbm.at[pl.ds(b, _CHD)], di, s_j)

    def iwait(j, p):
        si, di, _, _, s_i, s_j, _, _ = bufs[p]
        b = wbase + j * _CHD
        pltpu.make_async_copy(src_hbm.at[pl.ds(b, _CHD)], si, s_i).wait()
        pltpu.make_async_copy(dst_hbm.at[pl.ds(b, _CHD)], di, s_j).wait()

    def gstart(p):
        si, di, us, ud, _, _, s_g, s_h = bufs[p]
        pltpu.async_copy(u_hbm.at[si], us, s_g)
        pltpu.async_copy(u_hbm.at[di], ud, s_h)

    def gwait(p):
        si, di, us, ud, _, _, s_g, s_h = bufs[p]
        pltpu.make_async_copy(u_hbm.at[si], us, s_g).wait()
        pltpu.make_async_copy(u_hbm.at[di], ud, s_h).wait()

    def comp_scat(p):
        _, di, us, ud, _, _, _, _ = bufs[p]
        _sq_rows(us, ud, _CHD)
        pltpu.sync_copy(us, acc_sh.at[di], add=True)

    def half(j, x, y, next_idx):
        iwait(j + 1, y)
        gstart(y)
        gwait(x)
        comp_scat(x)
        if next_idx:
            istart(j + 2, x)

    istart(0, 0)
    iwait(0, 0)
    gstart(0)
    istart(1, 1)

    def body(k, cy):
        half(2 * k, 0, 1, True)
        half(2 * k + 1, 1, 0, True)
        return cy

    lax.fori_loop(0, (_NFD - 3) // 2, body, 0)
    half(_NFD - 3, 0, 1, True)
    half(_NFD - 2, 1, 0, False)
    gwait(0)
    tb = wbase + _NFD * _CHD
    pltpu.sync_copy(src_hbm.at[pl.ds(tb, _TLD)], si8)
    pltpu.sync_copy(dst_hbm.at[pl.ds(tb, _TLD)], di8)
    t1 = pltpu.async_copy(u_hbm.at[si8], us8, sem_t)
    t2 = pltpu.async_copy(u_hbm.at[di8], ud8, sem_t)
    comp_scat(0)
    t1.wait()
    t2.wait()
    _sq_rows(us8, ud8, _TLD)
    pltpu.sync_copy(us8, acc_sh.at[di8], add=True)
    plsc.subcore_barrier()

    def wb(off):
        pltpu.sync_copy(acc_sh.at[pl.ds(off, _WB)], zb_v)
        pltpu.sync_copy(zb_v, out_hbm.at[c, pl.ds(off, _WB)])

    _for_sub_blocks(s, wb)


@functools.cache
def _sc_diff():
    return pl.kernel(
        _sc_diff_body,
        out_type=jax.ShapeDtypeStruct((_NC, _N, _D), jnp.float32),
        mesh=_sc_mesh(),
        scratch_types=[
            pltpu.VMEM_SHARED((_N, _D), jnp.float32),
            pltpu.VMEM((_CHD,), jnp.int32),
            pltpu.VMEM((_CHD,), jnp.int32),
            pltpu.VMEM((_CHD,), jnp.int32),
            pltpu.VMEM((_CHD,), jnp.int32),
            pltpu.VMEM((_CHD, _D), jnp.float32),
            pltpu.VMEM((_CHD, _D), jnp.float32),
            pltpu.VMEM((_CHD, _D), jnp.float32),
            pltpu.VMEM((_CHD, _D), jnp.float32),
            pltpu.VMEM((_TLD,), jnp.int32),
            pltpu.VMEM((_TLD,), jnp.int32),
            pltpu.VMEM((_TLD, _D), jnp.float32),
            pltpu.VMEM((_TLD, _D), jnp.float32),
            pltpu.VMEM((_WB, _D), jnp.float32),
        ] + [pltpu.SemaphoreType.DMA] * 9,
    )


_BE = 4000


def _em_kernel(hs, hd, he, W1g, s1, b1p, eW2, eb2, Wm1g, s2, b2p, nW2, nb2,
               he_out, m_out):
    a = hs[...]
    b = hd[...]
    cc = he[...]
    sa = jnp.sum(a, 1, keepdims=True)
    qa = jnp.sum(a * a, 1, keepdims=True)
    inv = 1.0 / (2 * _D + _DE)
    mu = (sa + jnp.sum(b, 1, keepdims=True) + jnp.sum(cc, 1, keepdims=True)) * inv
    q = (qa + jnp.sum(b * b, 1, keepdims=True)
         + jnp.sum(cc * cc, 1, keepdims=True)) * inv
    r = lax.rsqrt(q - mu * mu + 1e-5)
    w = W1g[...]
    acc = (jnp.dot(a, w[:_D], preferred_element_type=jnp.float32)
           + jnp.dot(b, w[_D:2 * _D], preferred_element_type=jnp.float32)
           + jnp.dot(cc, w[2 * _D:], preferred_element_type=jnp.float32))
    t = jnp.maximum(r * acc - (r * mu) * s1[...] + b1p[...], 0.0)
    hen = cc + jnp.dot(t, eW2[...], preferred_element_type=jnp.float32) + eb2[...]
    he_out[...] = hen

    inv2 = 1.0 / (_D + _DE)
    mu2 = (sa + jnp.sum(hen, 1, keepdims=True)) * inv2
    q2 = (qa + jnp.sum(hen * hen, 1, keepdims=True)) * inv2
    r2 = lax.rsqrt(q2 - mu2 * mu2 + 1e-5)
    w2 = Wm1g[...]
    acc2 = (jnp.dot(a, w2[:_D], preferred_element_type=jnp.float32)
            + jnp.dot(hen, w2[_D:], preferred_element_type=jnp.float32))
    t2 = jnp.maximum(r2 * acc2 - (r2 * mu2) * s2[...] + b2p[...], 0.0)
    m_out[...] = jnp.dot(t2, nW2[...], preferred_element_type=jnp.float32) + nb2[...]


def _full(shape):
    return pl.BlockSpec(shape, lambda j: tuple(0 for _ in shape))


def _em_call(hs, hd, he, W1g, s1, b1p, eW2, eb2, Wm1g, s2, b2p, nW2, nb2):
    grid = _EH // _BE
    espec = lambda w: pl.BlockSpec((_BE, w), lambda j: (j, 0))
    return pl.pallas_call(
        _em_kernel,
        grid=(grid,),
        in_specs=[
            espec(_D), espec(_D), espec(_DE),
            _full(W1g.shape), _full(s1.shape), _full(b1p.shape),
            _full(eW2.shape), _full(eb2.shape),
            _full(Wm1g.shape), _full(s2.shape), _full(b2p.shape),
            _full(nW2.shape), _full(nb2.shape),
        ],
        out_specs=[espec(_DE), espec(_D)],
        out_shape=[jax.ShapeDtypeStruct((_EH, _DE), jnp.float32),
                   jax.ShapeDtypeStruct((_EH, _D), jnp.float32)],
    )(hs, hd, he, W1g, s1, b1p, eW2, eb2, Wm1g, s2, b2p, nW2, nb2)


_BN = 2000


def _u_kernel(h, aggPA, aggPB, degPA, degPB, Wu1, bu1, Wu2, bu2, u_out, deg_out):
    agg = (aggPA[0] + aggPA[1]) + (aggPB[0] + aggPB[1])
    deg = jnp.maximum((degPA[0] + degPA[1]) + (degPB[0] + degPB[1]), 1.0)
    aggn = agg / deg
    hh = h[...]
    w1 = Wu1[...]
    t = (jnp.dot(hh, w1[:_D], preferred_element_type=jnp.float32)
         + jnp.dot(aggn, w1[_D:], preferred_element_type=jnp.float32)
         + bu1[...])
    t = jnp.maximum(t, 0.0)
    u_out[...] = jnp.dot(t, Wu2[...], preferred_element_type=jnp.float32) + bu2[...]
    deg_out[...] = deg


def _u_call(h, aggPA, aggPB, degPA, degPB, Wu1, bu1, Wu2, bu2):
    grid = _N // _BN
    nspec = pl.BlockSpec((_BN, _D), lambda j: (j, 0))
    pspec = pl.BlockSpec((_NC, _BN, _D), lambda j: (0, j, 0))
    return pl.pallas_call(
        _u_kernel,
        grid=(grid,),
        in_specs=[
            nspec, pspec, pspec, pspec, pspec,
            _full(Wu1.shape), _full(bu1.shape), _full(Wu2.shape), _full(bu2.shape),
        ],
        out_specs=[nspec, nspec],
        out_shape=[jax.ShapeDtypeStruct((_N, _D), jnp.float32),
                   jax.ShapeDtypeStruct((_N, _D), jnp.float32)],
    )(h, aggPA, aggPB, degPA, degPB, Wu1, bu1, Wu2, bu2)


def _h_kernel(h, u, diffP, degb, out):
    dsum = diffP[0] + diffP[1]
    tau = jnp.tanh(dsum / degb[...])
    out[...] = (1.0 - tau) * h[...] + tau * u[...]


def _h_call(h, u, diffP, degb):
    grid = _N // _BN
    nspec = pl.BlockSpec((_BN, _D), lambda j: (j, 0))
    return pl.pallas_call(
        _h_kernel,
        grid=(grid,),
        in_specs=[nspec, nspec,
                  pl.BlockSpec((_NC, _BN, _D), lambda j: (0, j, 0)), nspec],
        out_specs=nspec,
        out_shape=jax.ShapeDtypeStruct((_N, _D), jnp.float32),
    )(h, u, diffP, degb)


def _f_kernel(h, pg, pb, lW, lb, mg, mb, loc_out, glob_out, acc):
    j = pl.program_id(0)
    hh = h[...]
    mu = jnp.mean(hh, 1, keepdims=True)
    hm = hh - mu
    var = jnp.mean(hm * hm, 1, keepdims=True)
    loc = hm * lax.rsqrt(var + 1e-5) * pg[...] + pb[...]
    loc_out[...] = loc

    @pl.when(j == 0)
    def _():
        acc[...] = jnp.zeros_like(acc)

    acc[...] += jnp.sum(loc, 0, keepdims=True)
    pooled = acc[...] * (1.0 / _N)
    g = jnp.dot(pooled, lW[...], preferred_element_type=jnp.float32) + lb[...]
    mu2 = jnp.mean(g, 1, keepdims=True)
    gm = g - mu2
    var2 = jnp.mean(gm * gm, 1, keepdims=True)
    glob_out[...] = gm * lax.rsqrt(var2 + 1e-5) * mg[...] + mb[...]


def _f_call(h, pg, pb, lW, lb, mg, mb):
    grid = _N // _BN
    return pl.pallas_call(
        _f_kernel,
        grid=(grid,),
        in_specs=[pl.BlockSpec((_BN, _D), lambda j: (j, 0)),
                  _full(pg.shape), _full(pb.shape), _full(lW.shape),
                  _full(lb.shape), _full(mg.shape), _full(mb.shape)],
        out_specs=[pl.BlockSpec((_BN, _D), lambda j: (j, 0)),
                   pl.BlockSpec((1, _D), lambda j: (0, 0))],
        out_shape=[jax.ShapeDtypeStruct((_N, _D), jnp.float32),
                   jax.ShapeDtypeStruct((1, _D), jnp.float32)],
        scratch_shapes=[pltpu.VMEM((1, _D), jnp.float32)],
    )(h, pg, pb, lW, lb, mg, mb)


def kernel(h, he, edge_index, eW1, eb1, eW2, eb2, eln_g, eln_b, nWm1, nbm1,
           nWm2, nbm2, nln_g, nln_b, nWu1, nbu1, nWu2, nbu2, pn_g, pn_b,
           lin_W, lin_b, mln_g, mln_b):
    src = edge_index[0]
    dst = edge_index[1]
    row = lambda v: v.reshape(1, -1)
    heA, heB = he[:_EH], he[_EH:]
    degPA = degPB = None
    for i in (0, 1):
        if i == 0:
            hsA, hdA, degPA = _sc_gather(0, True)(h, src, dst)
            hsB, hdB, degPB = _sc_gather(1, True)(h, src, dst)
        else:
            hsA, hdA = _sc_gather(0, False)(h, src, dst)
            hsB, hdB = _sc_gather(1, False)(h, src, dst)
        W1g = eW1[i] * eln_g[i][:, None]
        s1 = jnp.sum(W1g, 0)
        b1p = eb1[i] + eln_b[i] @ eW1[i]
        Wm1g = nWm1[i] * nln_g[i][:, None]
        s2 = jnp.sum(Wm1g, 0)
        b2p = nbm1[i] + nln_b[i] @ nWm1[i]
        ew = (W1g, row(s1), row(b1p), eW2[i], row(eb2[i]),
              Wm1g, row(s2), row(b2p), nWm2[i], row(nbm2[i]))
        heA, mA = _em_call(hsA, hdA, heA, *ew)
        heB, mB = _em_call(hsB, hdB, heB, *ew)
        aggPA = _sc_segsum(0)(mA, dst)
        aggPB = _sc_segsum(1)(mB, dst)
        u, degb = _u_call(h, aggPA, aggPB, degPA, degPB,
                          nWu1[i], row(nbu1[i]), nWu2[i], row(nbu2[i]))
        diffP = _sc_diff()(u, src, dst)
        h = _h_call(h, u, diffP, degb)
    return _f_call(h, row(pn_g), row(pn_b), lin_W, row(lin_b),
                   row(mln_g), row(mln_b))
